# baseline (speedup 1.0000x reference)
import functools

import jax
import jax.numpy as jnp
import numpy as np
from jax import lax
from jax.experimental import pallas as pl
from jax.experimental.pallas import tpu as pltpu
from jax.experimental.pallas import tpu_sc as plsc

NU = 10000
NB = 10000
D0 = 64
EU = 320000
EB = 160000
R_MIN = 1.0
R_MAX = 5.0

NC = 2
NS = 16
NW = NC * NS


def _leaky(x):
    return jnp.where(x > 0, x, 0.2 * x)


def _t1_body(s_ref, w1_ref, h1_ref):
    h1_ref[...] = lax.dot_general(s_ref[...], w1_ref[...],
                                  (((1,), (1,)), ((), ())),
                                  preferred_element_type=jnp.float32)


def _pdps_body(a2_ref, h1_ref, pdps_ref):
    pdps_ref[...] = lax.dot_general(a2_ref[...], h1_ref[...],
                                    (((1,), (1,)), ((), ())),
                                    preferred_element_type=jnp.float32)


def _t1(S, W1, a):
    N, SD = S.shape
    a2 = jnp.stack([a[:D0], a[D0:]])
    R = 2000
    h1 = pl.pallas_call(
        _t1_body,
        grid=(N // R,),
        in_specs=[
            pl.BlockSpec((R, SD), lambda i: (i, 0)),
            pl.BlockSpec((D0, SD), lambda i: (0, 0)),
        ],
        out_specs=pl.BlockSpec((R, D0), lambda i: (i, 0)),
        out_shape=jax.ShapeDtypeStruct((N, D0), jnp.float32),
    )(S, W1)
    pdps = pl.pallas_call(
        _pdps_body,
        out_shape=jax.ShapeDtypeStruct((2, N), jnp.float32),
    )(a2, h1)
    return h1, pdps


_B = 128
NUP = 10240


def _emit_rows(h1_ref, acc_s, den_s, src_i, dst_i, rows_v, w_v):
    pltpu.sync_copy(h1_ref.at[src_i], rows_v)

    def scale_row(e, _):
        idx_e = jnp.zeros((16,), jnp.int32) + e
        w16 = plsc.load_gather(w_v, [idx_e])
        for c in range(D0 // 16):
            rows_v[e, pl.ds(c * 16, 16)] = rows_v[e, pl.ds(c * 16, 16)] * w16
        return 0

    lax.fori_loop(0, _B, scale_row, 0, unroll=2)

    pltpu.sync_copy(rows_v, acc_s.at[dst_i], add=True)
    pltpu.sync_copy(w_v, den_s.at[dst_i], add=True)


def _edge_block(ei_ref, n_edges, h1_ref, pd_tab, ps_tab, acc_s, den_s,
                src_i, dst_i, rows_v, w_v, base, B, scale):
    pltpu.sync_copy(ei_ref.at[pl.ds(base, B)], src_i)
    pltpu.sync_copy(ei_ref.at[pl.ds(n_edges + base, B)], dst_i)

    def scores(i, _):
        s16 = src_i[pl.ds(i * 16, 16)]
        d16 = dst_i[pl.ds(i * 16, 16)]
        pdv = plsc.load_gather(pd_tab, [d16])
        psv = plsc.load_gather(ps_tab, [s16])
        e = _leaky(pdv + psv) * scale
        w_v[pl.ds(i * 16, 16)] = jnp.exp(e)
        return 0

    lax.fori_loop(0, B // 16, scores, 0, unroll=4)
    _emit_rows(h1_ref, acc_s, den_s, src_i, dst_i, rows_v, w_v)


def _part_copy(src, dst, sid):
    base = pl.multiple_of(sid * 640, 128)
    pltpu.sync_copy(src.at[pl.ds(base, 640)], dst.at[pl.ds(base, 640)])


def _su_kernel(ei_ref, h1_ref, pd_ref, ps_ref, z64_ref, z1_ref,
               acc_out, den_out,
               pd_tab, ps_tab, src_i, dst_i, rows_v, w_v, acc_s, den_s):
    cid = lax.axis_index("c")
    sid = lax.axis_index("s")
    wid = cid * NS + sid

    pltpu.sync_copy(pd_ref, pd_tab)
    pltpu.sync_copy(ps_ref, ps_tab)
    _part_copy(z64_ref, acc_s, sid)
    pltpu.sync_copy(z1_ref.at[pl.ds(pl.multiple_of(sid * 640, 128), 640)],
                    den_s.at[pl.ds(pl.multiple_of(sid * 640, 128), 640)])
    plsc.subcore_barrier()

    total_blk = EU // _B
    nbase = total_blk // NW
    extra = total_blk - nbase * NW
    nblk = nbase + jnp.where(wid < extra, 1, 0)
    sblk = nbase * wid + jnp.minimum(wid, extra)

    def blk(b, _):
        base = pl.multiple_of((sblk + b) * _B, _B)
        _edge_block(ei_ref, EU, h1_ref, pd_tab, ps_tab, acc_s, den_s,
                    src_i, dst_i, rows_v, w_v, base, _B, 1.0)
        return 0

    lax.fori_loop(0, nblk, blk, 0)

    plsc.subcore_barrier()
    _part_copy(acc_s, acc_out.at[cid], sid)
    base = pl.multiple_of(sid * 640, 128)
    obase = pl.multiple_of(cid * NUP + sid * 640, 128)
    pltpu.sync_copy(den_s.at[pl.ds(base, 640)],
                    den_out.at[pl.ds(obase, 640)])


def _su(edge_index, h1, pdps):
    mesh = plsc.VectorSubcoreMesh(core_axis_name="c", subcore_axis_name="s")
    z64 = jnp.zeros((NUP, D0), jnp.float32)
    z1 = jnp.zeros((NUP,), jnp.float32)
    pdp = jnp.pad(pdps, ((0, 0), (0, NUP - NU)))
    f = functools.partial(
        pl.kernel,
        mesh=mesh,
        compiler_params=pltpu.CompilerParams(needs_layout_passes=False,
                                             use_tc_tiling_on_sc=False),
        out_type=[
            jax.ShapeDtypeStruct((NC, NUP, D0), jnp.float32),
            jax.ShapeDtypeStruct((NC * NUP,), jnp.float32),
        ],
        scratch_types=[
            pltpu.VMEM((NUP,), jnp.float32),
            pltpu.VMEM((NUP,), jnp.float32),
            pltpu.VMEM((_B,), jnp.int32),
            pltpu.VMEM((_B,), jnp.int32),
            pltpu.VMEM((_B, D0), jnp.float32),
            pltpu.VMEM((_B,), jnp.float32),
            pltpu.VMEM_SHARED((NUP, D0), jnp.float32),
            pltpu.VMEM_SHARED((NUP,), jnp.float32),
        ],
    )(_su_kernel)
    acc, den = f(edge_index.reshape(-1), h1, pdp[0], pdp[1], z64, z1)
    return acc[:, :NU, :], den.reshape(NC, NUP)[:, :NU]


EB3 = 3 * EB
_MHALF = 2_000_000
_MTOT = 2 * _MHALF
_DUMP = 2048
_K1 = np.int32(-1640531527)
_K2 = np.int32(-2048144789)
_K3 = np.int32(-1028477371)
_MASK31 = np.int32(0x7FFFFFFF)
_TS = 16384


def _slot_of(key):
    return ((key * _K1) & _MASK31) % _MTOT


def _cls_of(key):
    return ((key * _K2) & _MASK31) >> 26


def _probe_of(key):
    return ((key * _K3) & _MASK31) % _TS


def _sb1_kernel(src_ref, dst_ref, zb_ref, cnt_out,
                src_i, dst_i, h_i, ones_v, cnt_s):
    cid = lax.axis_index("c")
    sid = lax.axis_index("s")
    iota = lax.iota(jnp.int32, 16)

    csz = (_MHALF + _DUMP) // NS
    cbase = pl.multiple_of(sid * csz, 8)
    for j in range(7):
        pltpu.sync_copy(zb_ref, cnt_s.at[pl.ds(cbase + j * 16384, 16384)])
    pltpu.sync_copy(zb_ref.at[pl.ds(0, csz - 7 * 16384)],
                    cnt_s.at[pl.ds(cbase + 7 * 16384, csz - 7 * 16384)])

    def ones(i, _):
        ones_v[pl.ds(i * 16, 16)] = jnp.zeros((16,), jnp.int32) + 1
        return 0

    lax.fori_loop(0, _B // 16, ones, 0)
    plsc.subcore_barrier()

    total_blk = EB3 // _B
    nbase = total_blk // NS
    extra = total_blk - nbase * NS
    nblk = nbase + jnp.where(sid < extra, 1, 0)
    sblk = nbase * sid + jnp.minimum(sid, extra)
    half_lo = cid * _MHALF

    def blk(b, _):
        gb = sblk + b
        base = pl.multiple_of(gb * _B, _B)
        pltpu.sync_copy(src_ref.at[pl.ds(base, _B)], src_i)
        pltpu.sync_copy(dst_ref.at[pl.ds(base, _B)], dst_i)

        def chunk(i, _):
            s16 = src_i[pl.ds(i * 16, 16)]
            d16 = dst_i[pl.ds(i * 16, 16)]
            h = _slot_of(s16 * NB + d16) - half_lo
            own = (h >= 0) & (h < _MHALF)
            dump = _MHALF + ((gb + i) % (_DUMP // 16)) * 16 + iota
            h_i[pl.ds(i * 16, 16)] = jnp.where(own, h, dump)
            return 0

        lax.fori_loop(0, _B // 16, chunk, 0, unroll=4)
        pltpu.sync_copy(ones_v, cnt_s.at[h_i], add=True)
        return 0

    lax.fori_loop(0, nblk, blk, 0)
    plsc.subcore_barrier()

    wsz = _MHALF // NS
    rbase = pl.multiple_of(sid * wsz, 8)
    pltpu.sync_copy(cnt_s.at[pl.ds(rbase, wsz)],
                    cnt_out.at[pl.ds(pl.multiple_of(cid * _MHALF + sid * wsz, 8),
                                     wsz)])


def _sb1(src_flat, dst_flat):
    mesh = plsc.VectorSubcoreMesh(core_axis_name="c", subcore_axis_name="s")
    zb = jnp.zeros((16384,), jnp.int32)
    f = functools.partial(
        pl.kernel,
        mesh=mesh,
        compiler_params=pltpu.CompilerParams(needs_layout_passes=False,
                                             use_tc_tiling_on_sc=False),
        out_type=[jax.ShapeDtypeStruct((_MTOT,), jnp.int32)],
        scratch_types=[
            pltpu.VMEM((_B,), jnp.int32),
            pltpu.VMEM((_B,), jnp.int32),
            pltpu.VMEM((_B,), jnp.int32),
            pltpu.VMEM((_B,), jnp.int32),
            pltpu.VMEM_SHARED((_MHALF + _DUMP,), jnp.int32),
        ],
    )(_sb1_kernel)
    [cnt] = f(src_flat, dst_flat, zb)
    return cnt


def _sb2_kernel(src_ref, dst_ref, cnt_ref, h1_ref, qd_ref, qs_ref, om_ref,
                z64_ref, z1_ref, acc_out, den_out, susp_out,
                qd_tab, qs_tab, om_tab, src_i, dst_i, h_i, cnt_i, susp_v,
                rows_v, w_v, acc_s, den_s):
    cid = lax.axis_index("c")
    sid = lax.axis_index("s")
    wid = cid * NS + sid
    iota = lax.iota(jnp.int32, 16)

    pltpu.sync_copy(qd_ref, qd_tab)
    pltpu.sync_copy(qs_ref, qs_tab)
    pltpu.sync_copy(om_ref, om_tab)
    _part_copy(z64_ref, acc_s, sid)
    pltpu.sync_copy(z1_ref.at[pl.ds(pl.multiple_of(sid * 640, 128), 640)],
                    den_s.at[pl.ds(pl.multiple_of(sid * 640, 128), 640)])
    plsc.subcore_barrier()

    total_blk = EB3 // _B
    nbase = total_blk // NW
    extra = total_blk - nbase * NW
    nblk = nbase + jnp.where(wid < extra, 1, 0)
    sblk = nbase * wid + jnp.minimum(wid, extra)

    def blk(b, _):
        gb = sblk + b
        base = pl.multiple_of(gb * _B, _B)
        g = gb // (EB // _B)
        omv = plsc.load_gather(om_tab, [jnp.zeros((16,), jnp.int32) + g])
        pltpu.sync_copy(src_ref.at[pl.ds(base, _B)], src_i)
        pltpu.sync_copy(dst_ref.at[pl.ds(base, _B)], dst_i)

        def hchunk(i, _):
            s16 = src_i[pl.ds(i * 16, 16)]
            d16 = dst_i[pl.ds(i * 16, 16)]
            h_i[pl.ds(i * 16, 16)] = _slot_of(s16 * NB + d16)
            return 0

        lax.fori_loop(0, _B // 16, hchunk, 0, unroll=4)
        pltpu.sync_copy(cnt_ref.at[h_i], cnt_i)

        def chunk(i, _):
            s16 = src_i[pl.ds(i * 16, 16)]
            d16 = dst_i[pl.ds(i * 16, 16)]
            c16 = cnt_i[pl.ds(i * 16, 16)]
            key = s16 * NB + d16
            qdv = plsc.load_gather(qd_tab, [d16])
            qsv = plsc.load_gather(qs_tab, [s16])
            expe = jnp.exp(omv * _leaky(qdv + qsv))
            fast = c16 == 1
            w_v[pl.ds(i * 16, 16)] = jnp.where(fast, expe, 0.0)
            dst_i[pl.ds(i * 16, 16)] = jnp.where(fast, d16, NU + 16 + iota)
            susp_v[pl.ds(i * 16, 16)] = jnp.where(fast, -1, key * 4 + g)
            return 0

        lax.fori_loop(0, _B // 16, chunk, 0, unroll=2)
        pltpu.sync_copy(susp_v, susp_out.at[pl.ds(base, _B)])
        _emit_rows(h1_ref, acc_s, den_s, src_i, dst_i, rows_v, w_v)
        return 0

    lax.fori_loop(0, nblk, blk, 0)

    plsc.subcore_barrier()
    _part_copy(acc_s, acc_out.at[cid], sid)
    base = pl.multiple_of(sid * 640, 128)
    obase = pl.multiple_of(cid * NUP + sid * 640, 128)
    pltpu.sync_copy(den_s.at[pl.ds(base, 640)], den_out.at[pl.ds(obase, 640)])


def _sb2(src_flat, dst_flat, cnt, h1b, qd, qs, om16):
    mesh = plsc.VectorSubcoreMesh(core_axis_name="c", subcore_axis_name="s")
    z64 = jnp.zeros((NUP, D0), jnp.float32)
    z1 = jnp.zeros((NUP,), jnp.float32)
    f = functools.partial(
        pl.kernel,
        mesh=mesh,
        compiler_params=pltpu.CompilerParams(needs_layout_passes=False,
                                             use_tc_tiling_on_sc=False),
        out_type=[
            jax.ShapeDtypeStruct((NC, NUP, D0), jnp.float32),
            jax.ShapeDtypeStruct((NC * NUP,), jnp.float32),
            jax.ShapeDtypeStruct((EB3,), jnp.int32),
        ],
        scratch_types=[
            pltpu.VMEM((NUP,), jnp.float32),
            pltpu.VMEM((NUP,), jnp.float32),
            pltpu.VMEM((16,), jnp.float32),
            pltpu.VMEM((_B,), jnp.int32),
            pltpu.VMEM((_B,), jnp.int32),
            pltpu.VMEM((_B,), jnp.int32),
            pltpu.VMEM((_B,), jnp.int32),
            pltpu.VMEM((_B,), jnp.int32),
            pltpu.VMEM((_B, D0), jnp.float32),
            pltpu.VMEM((_B,), jnp.float32),
            pltpu.VMEM_SHARED((NUP, D0), jnp.float32),
            pltpu.VMEM_SHARED((NUP,), jnp.float32),
        ],
    )(_sb2_kernel)
    return f(src_flat, dst_flat, cnt, h1b, qd, qs, om16, z64, z1)


_SCAN = 4096


def _sb3_kernel(susp_ref, h1_ref, qd_ref, qs_ref, om_ref, z64_ref, z1_ref,
                acc_out, den_out,
                qd_tab, qs_tab, om_tab, scan_v, src_i, dst_i, rows_v, w_v,
                tk, tw, acc_s, den_s):
    cid = lax.axis_index("c")
    sid = lax.axis_index("s")
    wid = cid * NS + sid
    iota = lax.iota(jnp.int32, 16)

    pltpu.sync_copy(qd_ref, qd_tab)
    pltpu.sync_copy(qs_ref, qs_tab)
    pltpu.sync_copy(om_ref, om_tab)
    _part_copy(z64_ref, acc_s, sid)
    pltpu.sync_copy(z1_ref.at[pl.ds(pl.multiple_of(sid * 640, 128), 640)],
                    den_s.at[pl.ds(pl.multiple_of(sid * 640, 128), 640)])

    def init_tab(i, _):
        tk[pl.ds(i * 16, 16)] = jnp.zeros((16,), jnp.int32) - 1
        tw[pl.ds(i * 16, 16)] = jnp.zeros((16,), jnp.float32)
        return 0

    lax.fori_loop(0, (_TS + 16) // 16, init_tab, 0)
    plsc.subcore_barrier()

    def probe_one(pk):
        key = pk >> 2
        g = pk & 3
        omv = plsc.load_gather(om_tab, [jnp.zeros((16,), jnp.int32) + g])
        h0 = _probe_of(key)

        def cond(h):
            tkh = jnp.max(plsc.load_gather(tk, [jnp.zeros((16,), jnp.int32) + h]))
            return (tkh != key) & (tkh != -1)

        def step(h):
            return (h + 1) % _TS

        h = lax.while_loop(cond, step, h0)
        hv = jnp.zeros((16,), jnp.int32) + h
        lane0 = iota == 0
        plsc.store_scatter(tk, [hv], jnp.zeros((16,), jnp.int32) + key, mask=lane0)
        plsc.addupdate_scatter(tw, [hv], omv, mask=lane0)

    nsc = EB3 // _SCAN
    tail = EB3 - nsc * _SCAN

    def scan_block(b, _):
        base = pl.multiple_of(b * _SCAN, 128)
        pltpu.sync_copy(susp_ref.at[pl.ds(base, _SCAN)], scan_v)

        def chunk(i, _):
            pkv = scan_v[pl.ds(i * 16, 16)]
            mine = (pkv >= 0) & (_cls_of(pkv >> 2) == wid)
            cnt = plsc.all_reduce_population_count(mine)

            @pl.when(cnt[0] > 0)
            def _():
                mine_i = jnp.where(mine, pkv, -1)
                for lane in range(16):
                    pk_l = mine_i[lane]

                    @pl.when(pk_l >= 0)
                    def _():
                        probe_one(pk_l)

            return 0

        lax.fori_loop(0, _SCAN // 16, chunk, 0, unroll=4)
        return 0

    lax.fori_loop(0, nsc, scan_block, 0)
    base = pl.multiple_of(nsc * _SCAN, 128)
    pltpu.sync_copy(susp_ref.at[pl.ds(base, tail)], scan_v.at[pl.ds(0, tail)])

    def tchunk(i, _):
        pkv = scan_v[pl.ds(i * 16, 16)]
        mine = (pkv >= 0) & (_cls_of(pkv >> 2) == wid)
        cnt = plsc.all_reduce_population_count(mine)

        @pl.when(cnt[0] > 0)
        def _():
            mine_i = jnp.where(mine, pkv, -1)
            for lane in range(16):
                pk_l = mine_i[lane]

                @pl.when(pk_l >= 0)
                def _():
                    probe_one(pk_l)

        return 0

    lax.fori_loop(0, tail // 16, tchunk, 0)

    def emit_block(tb, _):
        for i in range(_B // 16):
            key = tk[pl.ds(tb * _B + i * 16, 16)]
            wv = tw[pl.ds(tb * _B + i * 16, 16)]
            valid = key >= 0
            s16 = jnp.where(valid, key // NB, i * 16 + iota)
            d16 = key % NB
            qdv = plsc.load_gather(qd_tab, [jnp.where(valid, d16, 0)])
            qsv = plsc.load_gather(qs_tab, [s16])
            expe = jnp.where(valid, jnp.exp(wv * _leaky(qdv + qsv)), 0.0)
            src_i[pl.ds(i * 16, 16)] = s16
            dst_i[pl.ds(i * 16, 16)] = jnp.where(valid, d16, NU + 16 + iota)
            w_v[pl.ds(i * 16, 16)] = expe
        _emit_rows(h1_ref, acc_s, den_s, src_i, dst_i, rows_v, w_v)
        return 0

    lax.fori_loop(0, _TS // _B, emit_block, 0)

    plsc.subcore_barrier()
    _part_copy(acc_s, acc_out.at[cid], sid)
    base = pl.multiple_of(sid * 640, 128)
    obase = pl.multiple_of(cid * NUP + sid * 640, 128)
    pltpu.sync_copy(den_s.at[pl.ds(base, 640)], den_out.at[pl.ds(obase, 640)])


def _sb3(susp, h1b, qd, qs, om16):
    mesh = plsc.VectorSubcoreMesh(core_axis_name="c", subcore_axis_name="s")
    z64 = jnp.zeros((NUP, D0), jnp.float32)
    z1 = jnp.zeros((NUP,), jnp.float32)
    f = functools.partial(
        pl.kernel,
        mesh=mesh,
        compiler_params=pltpu.CompilerParams(needs_layout_passes=False,
                                             use_tc_tiling_on_sc=False),
        out_type=[
            jax.ShapeDtypeStruct((NC, NUP, D0), jnp.float32),
            jax.ShapeDtypeStruct((NC * NUP,), jnp.float32),
        ],
        scratch_types=[
            pltpu.VMEM((NUP,), jnp.float32),
            pltpu.VMEM((NUP,), jnp.float32),
            pltpu.VMEM((16,), jnp.float32),
            pltpu.VMEM((_SCAN,), jnp.int32),
            pltpu.VMEM((_B,), jnp.int32),
            pltpu.VMEM((_B,), jnp.int32),
            pltpu.VMEM((_B, D0), jnp.float32),
            pltpu.VMEM((_B,), jnp.float32),
            pltpu.VMEM((_TS + 16,), jnp.int32),
            pltpu.VMEM((_TS + 16,), jnp.float32),
            pltpu.VMEM_SHARED((NUP, D0), jnp.float32),
            pltpu.VMEM_SHARED((NUP,), jnp.float32),
        ],
    )(_sb3_kernel)
    return f(susp, h1b, qd, qs, om16, z64, z1)


def _t2_body(acc_ref, den_ref, s_ref, w2_ref, w2s_ref, b1_ref, w3_ref,
             h4_ref, out_ref):
    P = acc_ref.shape[0]
    num = acc_ref[0]
    den = den_ref[0, 0] + 1e-16
    for p in range(1, P):
        num = num + acc_ref[p]
        den = den + den_ref[0, p]
    h2 = num / den[:, None]
    h3 = (lax.dot_general(h2, w2_ref[...], (((1,), (1,)), ((), ())),
                          preferred_element_type=jnp.float32)
          + lax.dot_general(s_ref[...], w2s_ref[...], (((1,), (1,)), ((), ())),
                            preferred_element_type=jnp.float32)
          + b1_ref[...])
    h3 = jnp.where(h3 > 0, h3, jnp.exp(jnp.minimum(h3, 0.0)) - 1.0)
    u = lax.dot_general(h3, w3_ref[...], (((1,), (1,)), ((), ())),
                        preferred_element_type=jnp.float32)
    out_ref[...] = jnp.maximum(u, 0.0) + h4_ref[...]


def _t2(acc, den, S, W2, W2s, b1, W3, H4):
    N, SD = S.shape
    D1 = W2.shape[0]
    P = acc.shape[0]
    R = 2000
    den_r = den.reshape(P, N // R, R).swapaxes(0, 1)
    return pl.pallas_call(
        _t2_body,
        grid=(N // R,),
        in_specs=[
            pl.BlockSpec((P, R, D0), lambda i: (0, i, 0)),
            pl.BlockSpec((1, P, R), lambda i: (i, 0, 0)),
            pl.BlockSpec((R, SD), lambda i: (i, 0)),
            pl.BlockSpec((D1, D0), lambda i: (0, 0)),
            pl.BlockSpec((D1, SD), lambda i: (0, 0)),
            pl.BlockSpec((D1,), lambda i: (0,)),
            pl.BlockSpec((D0, D1), lambda i: (0, 0)),
            pl.BlockSpec((R, D0), lambda i: (i, 0)),
        ],
        out_specs=pl.BlockSpec((R, D0), lambda i: (i, 0)),
        out_shape=jax.ShapeDtypeStruct((N, D0), jnp.float32),
    )(acc, den_r, S, W2, W2s, b1, W3, H4)


def kernel(S_u, S_b, edge_index_u, edge_index_b0, edge_index_b1, edge_index_b2,
           user_idx, biz_idx, W1_u, W1_b, a_u, a_b, omega, W2_u, W2_us, b1_u,
           W2_b, W2_bs, b1_b, W3_u, W3_b, H4_u, H4_b, bias_u_w, bias_b_w,
           bias_global):
    H1_u, pdps_u = _t1(S_u, W1_u, a_u)
    H1_b, pdps_b = _t1(S_b, W1_b, a_b)

    acc_u, den_u = _su(edge_index_u, H1_u, pdps_u)
    U_all = _t2(acc_u, den_u, S_u, W2_u, W2_us, b1_u, W3_u, H4_u)

    omega_s = jax.nn.softmax(omega)
    om16 = jnp.zeros((16,), jnp.float32).at[:3].set(omega_s)
    src_flat = jnp.concatenate([edge_index_b0[0], edge_index_b1[0],
                                edge_index_b2[0]])
    dst_flat = jnp.concatenate([edge_index_b0[1], edge_index_b1[1],
                                edge_index_b2[1]])
    pdp_b = jnp.pad(pdps_b, ((0, 0), (0, NUP - NU)))
    qd, qs = pdp_b[0], pdp_b[1]
    cnt = _sb1(src_flat, dst_flat)
    acc2, den2, susp = _sb2(src_flat, dst_flat, cnt, H1_b, qd, qs, om16)
    acc3, den3 = _sb3(susp, H1_b, qd, qs, om16)
    accb = jnp.concatenate([acc2, acc3])[:, :NU, :]
    denb = jnp.concatenate([den2, den3]).reshape(4, NUP)[:, :NU]
    B_all = _t2(accb, denb, S_b, W2_b, W2_bs, b1_b, W3_b, H4_b)

    U_q = U_all[user_idx]
    B_q = B_all[biz_idx]
    logit = ((U_q * B_q).sum(axis=1) + bias_u_w[user_idx, 0]
             + bias_b_w[biz_idx, 0] + bias_global[0])
    pred = (R_MAX - R_MIN) * jax.nn.sigmoid(logit) + R_MIN
    return (pred, U_all, B_all)


# speedup vs baseline: 1.6806x; 1.6806x over previous
import functools

import jax
import jax.numpy as jnp
import numpy as np
from jax import lax
from jax.experimental import pallas as pl
from jax.experimental.pallas import tpu as pltpu
from jax.experimental.pallas import tpu_sc as plsc

NU = 10000
NB = 10000
D0 = 64
EU = 320000
EB = 160000
R_MIN = 1.0
R_MAX = 5.0

NC = 2
NS = 16
NW = NC * NS


def _leaky(x):
    return jnp.where(x > 0, x, 0.2 * x)


def _t1_body(s_ref, w1_ref, h1_ref):
    h1_ref[...] = lax.dot_general(s_ref[...], w1_ref[...],
                                  (((1,), (1,)), ((), ())),
                                  preferred_element_type=jnp.float32)


def _pdps_body(a2_ref, h1_ref, pdps_ref):
    pdps_ref[...] = lax.dot_general(a2_ref[...], h1_ref[...],
                                    (((1,), (1,)), ((), ())),
                                    preferred_element_type=jnp.float32)


def _t1(S, W1, a):
    N, SD = S.shape
    a2 = jnp.stack([a[:D0], a[D0:]])
    R = 2000
    h1 = pl.pallas_call(
        _t1_body,
        grid=(N // R,),
        in_specs=[
            pl.BlockSpec((R, SD), lambda i: (i, 0)),
            pl.BlockSpec((D0, SD), lambda i: (0, 0)),
        ],
        out_specs=pl.BlockSpec((R, D0), lambda i: (i, 0)),
        out_shape=jax.ShapeDtypeStruct((N, D0), jnp.float32),
    )(S, W1)
    pdps = pl.pallas_call(
        _pdps_body,
        out_shape=jax.ShapeDtypeStruct((2, N), jnp.float32),
    )(a2, h1)
    return h1, pdps


_B = 128
NUP = 10240


def _emit_rows(h1_ref, acc_s, den_s, src_i, dst_i, rows_v, w_v):
    pltpu.sync_copy(h1_ref.at[src_i], rows_v)

    def scale_row(e, _):
        idx_e = jnp.zeros((16,), jnp.int32) + e
        w16 = plsc.load_gather(w_v, [idx_e])
        for c in range(D0 // 16):
            rows_v[e, pl.ds(c * 16, 16)] = rows_v[e, pl.ds(c * 16, 16)] * w16
        return 0

    lax.fori_loop(0, _B, scale_row, 0, unroll=2)

    pltpu.sync_copy(rows_v, acc_s.at[dst_i], add=True)
    pltpu.sync_copy(w_v, den_s.at[dst_i], add=True)


def _edge_block(ei_ref, n_edges, h1_ref, pd_tab, ps_tab, acc_s, den_s,
                src_i, dst_i, rows_v, w_v, base, B, scale):
    pltpu.sync_copy(ei_ref.at[pl.ds(base, B)], src_i)
    pltpu.sync_copy(ei_ref.at[pl.ds(n_edges + base, B)], dst_i)

    def scores(i, _):
        s16 = src_i[pl.ds(i * 16, 16)]
        d16 = dst_i[pl.ds(i * 16, 16)]
        pdv = plsc.load_gather(pd_tab, [d16])
        psv = plsc.load_gather(ps_tab, [s16])
        e = _leaky(pdv + psv) * scale
        w_v[pl.ds(i * 16, 16)] = jnp.exp(e)
        return 0

    lax.fori_loop(0, B // 16, scores, 0, unroll=4)
    _emit_rows(h1_ref, acc_s, den_s, src_i, dst_i, rows_v, w_v)


def _part_copy(src, dst, sid):
    base = pl.multiple_of(sid * 640, 128)
    pltpu.sync_copy(src.at[pl.ds(base, 640)], dst.at[pl.ds(base, 640)])


def _su_kernel(ei_ref, h1_ref, pd_ref, ps_ref, z64_ref, z1_ref,
               acc_out, den_out,
               pd_tab, ps_tab, src_i, dst_i, rows_v, w_v, acc_s, den_s):
    cid = lax.axis_index("c")
    sid = lax.axis_index("s")
    wid = cid * NS + sid

    pltpu.sync_copy(pd_ref, pd_tab)
    pltpu.sync_copy(ps_ref, ps_tab)
    _part_copy(z64_ref, acc_s, sid)
    pltpu.sync_copy(z1_ref.at[pl.ds(pl.multiple_of(sid * 640, 128), 640)],
                    den_s.at[pl.ds(pl.multiple_of(sid * 640, 128), 640)])
    plsc.subcore_barrier()

    total_blk = EU // _B
    nbase = total_blk // NW
    extra = total_blk - nbase * NW
    nblk = nbase + jnp.where(wid < extra, 1, 0)
    sblk = nbase * wid + jnp.minimum(wid, extra)

    def blk(b, _):
        base = pl.multiple_of((sblk + b) * _B, _B)
        _edge_block(ei_ref, EU, h1_ref, pd_tab, ps_tab, acc_s, den_s,
                    src_i, dst_i, rows_v, w_v, base, _B, 1.0)
        return 0

    lax.fori_loop(0, nblk, blk, 0)

    plsc.subcore_barrier()
    _part_copy(acc_s, acc_out.at[cid], sid)
    base = pl.multiple_of(sid * 640, 128)
    obase = pl.multiple_of(cid * NUP + sid * 640, 128)
    pltpu.sync_copy(den_s.at[pl.ds(base, 640)],
                    den_out.at[pl.ds(obase, 640)])


def _su(edge_index, h1, pdps):
    mesh = plsc.VectorSubcoreMesh(core_axis_name="c", subcore_axis_name="s")
    z64 = jnp.zeros((NUP, D0), jnp.float32)
    z1 = jnp.zeros((NUP,), jnp.float32)
    pdp = jnp.pad(pdps, ((0, 0), (0, NUP - NU)))
    f = functools.partial(
        pl.kernel,
        mesh=mesh,
        compiler_params=pltpu.CompilerParams(needs_layout_passes=False,
                                             use_tc_tiling_on_sc=False),
        out_type=[
            jax.ShapeDtypeStruct((NC, NUP, D0), jnp.float32),
            jax.ShapeDtypeStruct((NC * NUP,), jnp.float32),
        ],
        scratch_types=[
            pltpu.VMEM((NUP,), jnp.float32),
            pltpu.VMEM((NUP,), jnp.float32),
            pltpu.VMEM((_B,), jnp.int32),
            pltpu.VMEM((_B,), jnp.int32),
            pltpu.VMEM((_B, D0), jnp.float32),
            pltpu.VMEM((_B,), jnp.float32),
            pltpu.VMEM_SHARED((NUP, D0), jnp.float32),
            pltpu.VMEM_SHARED((NUP,), jnp.float32),
        ],
    )(_su_kernel)
    acc, den = f(edge_index.reshape(-1), h1, pdp[0], pdp[1], z64, z1)
    return acc[:, :NU, :], den.reshape(NC, NUP)[:, :NU]


EB3 = 3 * EB
_MHALF = 2_000_000
_MTOT = 2 * _MHALF
_DUMP = 2048
_K1 = np.int32(-1640531527)
_K2 = np.int32(-2048144789)
_K3 = np.int32(-1028477371)
_MASK31 = np.int32(0x7FFFFFFF)
_TS = 16384


def _slot_of(key):
    return ((key * _K1) & _MASK31) % _MTOT


def _cls_of(key):
    return ((key * _K2) & _MASK31) >> 26


def _probe_of(key):
    return ((key * _K3) & _MASK31) % _TS


def _sb1_kernel(src_ref, dst_ref, zb_ref, cnt_out,
                src_i, dst_i, h_i, ones_v, cnt_s):
    cid = lax.axis_index("c")
    sid = lax.axis_index("s")
    iota = lax.iota(jnp.int32, 16)

    csz = (_MHALF + _DUMP) // NS
    cbase = pl.multiple_of(sid * csz, 8)
    for j in range(7):
        pltpu.sync_copy(zb_ref, cnt_s.at[pl.ds(cbase + j * 16384, 16384)])
    pltpu.sync_copy(zb_ref.at[pl.ds(0, csz - 7 * 16384)],
                    cnt_s.at[pl.ds(cbase + 7 * 16384, csz - 7 * 16384)])

    def ones(i, _):
        ones_v[pl.ds(i * 16, 16)] = jnp.zeros((16,), jnp.int32) + 1
        return 0

    lax.fori_loop(0, _B // 16, ones, 0)
    plsc.subcore_barrier()

    total_blk = EB3 // _B
    nbase = total_blk // NS
    extra = total_blk - nbase * NS
    nblk = nbase + jnp.where(sid < extra, 1, 0)
    sblk = nbase * sid + jnp.minimum(sid, extra)
    half_lo = cid * _MHALF

    def blk(b, _):
        gb = sblk + b
        base = pl.multiple_of(gb * _B, _B)
        pltpu.sync_copy(src_ref.at[pl.ds(base, _B)], src_i)
        pltpu.sync_copy(dst_ref.at[pl.ds(base, _B)], dst_i)

        def chunk(i, _):
            s16 = src_i[pl.ds(i * 16, 16)]
            d16 = dst_i[pl.ds(i * 16, 16)]
            h = _slot_of(s16 * NB + d16) - half_lo
            own = (h >= 0) & (h < _MHALF)
            dump = _MHALF + ((gb + i) % (_DUMP // 16)) * 16 + iota
            h_i[pl.ds(i * 16, 16)] = jnp.where(own, h, dump)
            return 0

        lax.fori_loop(0, _B // 16, chunk, 0, unroll=4)
        pltpu.sync_copy(ones_v, cnt_s.at[h_i], add=True)
        return 0

    lax.fori_loop(0, nblk, blk, 0)
    plsc.subcore_barrier()

    wsz = _MHALF // NS
    rbase = pl.multiple_of(sid * wsz, 8)
    pltpu.sync_copy(cnt_s.at[pl.ds(rbase, wsz)],
                    cnt_out.at[pl.ds(pl.multiple_of(cid * _MHALF + sid * wsz, 8),
                                     wsz)])


def _sb1(src_flat, dst_flat):
    mesh = plsc.VectorSubcoreMesh(core_axis_name="c", subcore_axis_name="s")
    zb = jnp.zeros((16384,), jnp.int32)
    f = functools.partial(
        pl.kernel,
        mesh=mesh,
        compiler_params=pltpu.CompilerParams(needs_layout_passes=False,
                                             use_tc_tiling_on_sc=False),
        out_type=[jax.ShapeDtypeStruct((_MTOT,), jnp.int32)],
        scratch_types=[
            pltpu.VMEM((_B,), jnp.int32),
            pltpu.VMEM((_B,), jnp.int32),
            pltpu.VMEM((_B,), jnp.int32),
            pltpu.VMEM((_B,), jnp.int32),
            pltpu.VMEM_SHARED((_MHALF + _DUMP,), jnp.int32),
        ],
    )(_sb1_kernel)
    [cnt] = f(src_flat, dst_flat, zb)
    return cnt


_CAP = 15104


def _sb2_kernel(src_ref, dst_ref, cnt_ref, h1_ref, qd_ref, qs_ref, om_ref,
                z64_ref, z1_ref, acc_out, den_out, susp_out, scnt_out,
                qd_tab, qs_tab, om_tab, src_i, dst_i, h_i, cnt_i, stage,
                cbuf, rows_v, w_v, acc_s, den_s):
    cid = lax.axis_index("c")
    sid = lax.axis_index("s")
    wid = cid * NS + sid
    iota = lax.iota(jnp.int32, 16)

    pltpu.sync_copy(qd_ref, qd_tab)
    pltpu.sync_copy(qs_ref, qs_tab)
    pltpu.sync_copy(om_ref, om_tab)
    _part_copy(z64_ref, acc_s, sid)
    pltpu.sync_copy(z1_ref.at[pl.ds(pl.multiple_of(sid * 640, 128), 640)],
                    den_s.at[pl.ds(pl.multiple_of(sid * 640, 128), 640)])
    plsc.subcore_barrier()

    total_blk = EB3 // _B
    nbase = total_blk // NW
    extra = total_blk - nbase * NW
    nblk = nbase + jnp.where(wid < extra, 1, 0)
    sblk = nbase * wid + jnp.minimum(wid, extra)

    def blk(b, carry):
        cur, nout = carry
        gb = sblk + b
        base = pl.multiple_of(gb * _B, _B)
        g = gb // (EB // _B)
        omv = plsc.load_gather(om_tab, [jnp.zeros((16,), jnp.int32) + g])
        pltpu.sync_copy(src_ref.at[pl.ds(base, _B)], src_i)
        pltpu.sync_copy(dst_ref.at[pl.ds(base, _B)], dst_i)

        def hchunk(i, _):
            s16 = src_i[pl.ds(i * 16, 16)]
            d16 = dst_i[pl.ds(i * 16, 16)]
            h_i[pl.ds(i * 16, 16)] = _slot_of(s16 * NB + d16)
            return 0

        lax.fori_loop(0, _B // 16, hchunk, 0, unroll=4)
        pltpu.sync_copy(cnt_ref.at[h_i], cnt_i)

        def chunk(i, cur):
            s16 = src_i[pl.ds(i * 16, 16)]
            d16 = dst_i[pl.ds(i * 16, 16)]
            c16 = cnt_i[pl.ds(i * 16, 16)]
            key = s16 * NB + d16
            qdv = plsc.load_gather(qd_tab, [d16])
            qsv = plsc.load_gather(qs_tab, [s16])
            expe = jnp.exp(omv * _leaky(qdv + qsv))
            fast = c16 == 1
            w_v[pl.ds(i * 16, 16)] = jnp.where(fast, expe, 0.0)
            dst_i[pl.ds(i * 16, 16)] = jnp.where(fast, d16, NU + 16 + iota)
            susp = jnp.logical_not(fast)
            plsc.store_compressed(stage.at[pl.ds(cur, 16)], key * 4 + g,
                                  mask=susp)
            return cur + plsc.all_reduce_population_count(susp)[0]

        cur = lax.fori_loop(0, _B // 16, chunk, cur)
        _emit_rows(h1_ref, acc_s, den_s, src_i, dst_i, rows_v, w_v)

        do_flush = cur >= _B

        @pl.when(do_flush)
        def _():
            obase = pl.multiple_of(wid * _CAP + nout, _B)
            pltpu.sync_copy(stage.at[pl.ds(0, _B)],
                            susp_out.at[pl.ds(obase, _B)])
            for j in range(_B // 16):
                stage[pl.ds(j * 16, 16)] = stage[pl.ds(_B + j * 16, 16)]

        cur = jnp.where(do_flush, cur - _B, cur)
        nout = jnp.where(do_flush, nout + _B, nout)
        return (cur, nout)

    cur, nout = lax.fori_loop(0, nblk, blk,
                              (jnp.int32(0), jnp.int32(0)))

    @pl.when(cur > 0)
    def _():
        obase = pl.multiple_of(wid * _CAP + nout, _B)
        pltpu.sync_copy(stage.at[pl.ds(0, _B)], susp_out.at[pl.ds(obase, _B)])

    total = cur + nout
    plsc.store_scatter(cbuf, [iota * 0], jnp.zeros((16,), jnp.int32) + total,
                       mask=iota == 0)
    pltpu.sync_copy(cbuf.at[pl.ds(0, 8)],
                    scnt_out.at[pl.ds(pl.multiple_of(wid * 8, 8), 8)])

    plsc.subcore_barrier()
    _part_copy(acc_s, acc_out.at[cid], sid)
    base = pl.multiple_of(sid * 640, 128)
    obase = pl.multiple_of(cid * NUP + sid * 640, 128)
    pltpu.sync_copy(den_s.at[pl.ds(base, 640)], den_out.at[pl.ds(obase, 640)])


def _sb2(src_flat, dst_flat, cnt, h1b, qd, qs, om16):
    mesh = plsc.VectorSubcoreMesh(core_axis_name="c", subcore_axis_name="s")
    z64 = jnp.zeros((NUP, D0), jnp.float32)
    z1 = jnp.zeros((NUP,), jnp.float32)
    f = functools.partial(
        pl.kernel,
        mesh=mesh,
        compiler_params=pltpu.CompilerParams(needs_layout_passes=False,
                                             use_tc_tiling_on_sc=False),
        out_type=[
            jax.ShapeDtypeStruct((NC, NUP, D0), jnp.float32),
            jax.ShapeDtypeStruct((NC * NUP,), jnp.float32),
            jax.ShapeDtypeStruct((NW * _CAP,), jnp.int32),
            jax.ShapeDtypeStruct((NW * 8,), jnp.int32),
        ],
        scratch_types=[
            pltpu.VMEM((NUP,), jnp.float32),
            pltpu.VMEM((NUP,), jnp.float32),
            pltpu.VMEM((16,), jnp.float32),
            pltpu.VMEM((_B,), jnp.int32),
            pltpu.VMEM((_B,), jnp.int32),
            pltpu.VMEM((_B,), jnp.int32),
            pltpu.VMEM((_B,), jnp.int32),
            pltpu.VMEM((2 * _B + 16,), jnp.int32),
            pltpu.VMEM((16,), jnp.int32),
            pltpu.VMEM((_B, D0), jnp.float32),
            pltpu.VMEM((_B,), jnp.float32),
            pltpu.VMEM_SHARED((NUP, D0), jnp.float32),
            pltpu.VMEM_SHARED((NUP,), jnp.float32),
        ],
    )(_sb2_kernel)
    return f(src_flat, dst_flat, cnt, h1b, qd, qs, om16, z64, z1)


_SCAN = 4096


def _sb3_kernel(susp_ref, cnt_ref, h1_ref, qd_ref, qs_ref, om_ref,
                z64_ref, z1_ref, acc_out, den_out,
                qd_tab, qs_tab, om_tab, scan_v, cnts_v, src_i, dst_i,
                rows_v, w_v, tk, tw, touched, scnt_ref, acc_s, den_s):
    cid = lax.axis_index("c")
    sid = lax.axis_index("s")
    wid = cid * NS + sid
    iota = lax.iota(jnp.int32, 16)

    pltpu.sync_copy(qd_ref, qd_tab)
    pltpu.sync_copy(qs_ref, qs_tab)
    pltpu.sync_copy(om_ref, om_tab)
    _part_copy(z64_ref, acc_s, sid)
    pltpu.sync_copy(z1_ref.at[pl.ds(pl.multiple_of(sid * 640, 128), 640)],
                    den_s.at[pl.ds(pl.multiple_of(sid * 640, 128), 640)])

    def init_tab(i, _):
        tk[pl.ds(i * 16, 16)] = jnp.zeros((16,), jnp.int32) - 1
        tw[pl.ds(i * 16, 16)] = jnp.zeros((16,), jnp.float32)
        return 0

    lax.fori_loop(0, (_TS + 16) // 16, init_tab, 0)
    scnt_ref[0] = jnp.int32(0)
    plsc.subcore_barrier()

    def probe_one(pk):
        key = pk >> 2
        g = pk & 3
        omv = plsc.load_gather(om_tab, [jnp.zeros((16,), jnp.int32) + g])
        h0 = _probe_of(key)

        def cond(h):
            tkh = jnp.max(plsc.load_gather(tk, [jnp.zeros((16,), jnp.int32) + h]))
            return (tkh != key) & (tkh != -1)

        def step(h):
            return (h + 1) % _TS

        h = lax.while_loop(cond, step, h0)
        hv = jnp.zeros((16,), jnp.int32) + h
        lane0 = iota == 0
        tkh = jnp.max(plsc.load_gather(tk, [hv]))
        plsc.store_scatter(tk, [hv], jnp.zeros((16,), jnp.int32) + key, mask=lane0)
        plsc.addupdate_scatter(tw, [hv], omv, mask=lane0)

        @pl.when(tkh != key)
        def _():
            cur = scnt_ref[0]
            plsc.store_scatter(touched, [jnp.zeros((16,), jnp.int32) + cur],
                               hv, mask=lane0)
            scnt_ref[0] = cur + 1

    pltpu.sync_copy(cnt_ref, cnts_v.at[pl.ds(0, NW * 8)])

    def region(r, _):
        c_r = cnts_v[pl.ds(r * 8, 16)][0]
        nb = (c_r + _SCAN - 1) // _SCAN

        def sblock(b, _):
            base = pl.multiple_of(r * _CAP + b * _SCAN, 128)
            pltpu.sync_copy(susp_ref.at[pl.ds(base, _SCAN)], scan_v)
            rem = c_r - b * _SCAN
            nch = (jnp.minimum(rem, _SCAN) + 15) // 16

            def chunk(i, _):
                pkv = scan_v[pl.ds(i * 16, 16)]
                vmask = (i * 16 + iota) < rem
                mine = vmask & (_cls_of(pkv >> 2) == wid)
                cnt = plsc.all_reduce_population_count(mine)

                @pl.when(cnt[0] > 0)
                def _():
                    mine_i = jnp.where(mine, pkv, -1)
                    for lane in range(16):
                        pk_l = mine_i[lane]

                        @pl.when(pk_l >= 0)
                        def _():
                            probe_one(pk_l)

                return 0

            lax.fori_loop(0, nch, chunk, 0)
            return 0

        lax.fori_loop(0, nb, sblock, 0)
        return 0

    lax.fori_loop(0, NW, region, 0)

    total = scnt_ref[0]
    nb_e = (total + _B - 1) // _B

    def emit_block(tb, _):
        for i in range(_B // 16):
            pos = tb * _B + i * 16
            hidx = touched[pl.ds(pos, 16)]
            hid = jnp.where(pos + iota < total, hidx, _TS)
            key = plsc.load_gather(tk, [hid])
            wv = plsc.load_gather(tw, [hid])
            valid = key >= 0
            s16 = jnp.where(valid, key // NB, i * 16 + iota)
            d16 = key % NB
            qdv = plsc.load_gather(qd_tab, [jnp.where(valid, d16, 0)])
            qsv = plsc.load_gather(qs_tab, [s16])
            expe = jnp.where(valid, jnp.exp(wv * _leaky(qdv + qsv)), 0.0)
            src_i[pl.ds(i * 16, 16)] = s16
            dst_i[pl.ds(i * 16, 16)] = jnp.where(valid, d16, NU + 16 + iota)
            w_v[pl.ds(i * 16, 16)] = expe
        _emit_rows(h1_ref, acc_s, den_s, src_i, dst_i, rows_v, w_v)
        return 0

    lax.fori_loop(0, nb_e, emit_block, 0)

    plsc.subcore_barrier()
    _part_copy(acc_s, acc_out.at[cid], sid)
    base = pl.multiple_of(sid * 640, 128)
    obase = pl.multiple_of(cid * NUP + sid * 640, 128)
    pltpu.sync_copy(den_s.at[pl.ds(base, 640)], den_out.at[pl.ds(obase, 640)])


def _sb3(susp, scnt, h1b, qd, qs, om16):
    mesh = plsc.VectorSubcoreMesh(core_axis_name="c", subcore_axis_name="s")
    z64 = jnp.zeros((NUP, D0), jnp.float32)
    z1 = jnp.zeros((NUP,), jnp.float32)
    f = functools.partial(
        pl.kernel,
        mesh=mesh,
        compiler_params=pltpu.CompilerParams(needs_layout_passes=False,
                                             use_tc_tiling_on_sc=False),
        out_type=[
            jax.ShapeDtypeStruct((NC, NUP, D0), jnp.float32),
            jax.ShapeDtypeStruct((NC * NUP,), jnp.float32),
        ],
        scratch_types=[
            pltpu.VMEM((NUP,), jnp.float32),
            pltpu.VMEM((NUP,), jnp.float32),
            pltpu.VMEM((16,), jnp.float32),
            pltpu.VMEM((_SCAN,), jnp.int32),
            pltpu.VMEM((NW * 8 + 16,), jnp.int32),
            pltpu.VMEM((_B,), jnp.int32),
            pltpu.VMEM((_B,), jnp.int32),
            pltpu.VMEM((_B, D0), jnp.float32),
            pltpu.VMEM((_B,), jnp.float32),
            pltpu.VMEM((_TS + 16,), jnp.int32),
            pltpu.VMEM((_TS + 16,), jnp.float32),
            pltpu.VMEM((_TS,), jnp.int32),
            pltpu.SMEM((8,), jnp.int32),
            pltpu.VMEM_SHARED((NUP, D0), jnp.float32),
            pltpu.VMEM_SHARED((NUP,), jnp.float32),
        ],
    )(_sb3_kernel)
    return f(susp, scnt, h1b, qd, qs, om16, z64, z1)


def _t2_body(acc_ref, den_ref, s_ref, w2_ref, w2s_ref, b1_ref, w3_ref,
             h4_ref, out_ref):
    P = acc_ref.shape[0]
    num = acc_ref[0]
    den = den_ref[0, 0] + 1e-16
    for p in range(1, P):
        num = num + acc_ref[p]
        den = den + den_ref[0, p]
    h2 = num / den[:, None]
    h3 = (lax.dot_general(h2, w2_ref[...], (((1,), (1,)), ((), ())),
                          preferred_element_type=jnp.float32)
          + lax.dot_general(s_ref[...], w2s_ref[...], (((1,), (1,)), ((), ())),
                            preferred_element_type=jnp.float32)
          + b1_ref[...])
    h3 = jnp.where(h3 > 0, h3, jnp.exp(jnp.minimum(h3, 0.0)) - 1.0)
    u = lax.dot_general(h3, w3_ref[...], (((1,), (1,)), ((), ())),
                        preferred_element_type=jnp.float32)
    out_ref[...] = jnp.maximum(u, 0.0) + h4_ref[...]


def _t2(acc, den, S, W2, W2s, b1, W3, H4):
    N, SD = S.shape
    D1 = W2.shape[0]
    P = acc.shape[0]
    R = 2000
    den_r = den.reshape(P, N // R, R).swapaxes(0, 1)
    return pl.pallas_call(
        _t2_body,
        grid=(N // R,),
        in_specs=[
            pl.BlockSpec((P, R, D0), lambda i: (0, i, 0)),
            pl.BlockSpec((1, P, R), lambda i: (i, 0, 0)),
            pl.BlockSpec((R, SD), lambda i: (i, 0)),
            pl.BlockSpec((D1, D0), lambda i: (0, 0)),
            pl.BlockSpec((D1, SD), lambda i: (0, 0)),
            pl.BlockSpec((D1,), lambda i: (0,)),
            pl.BlockSpec((D0, D1), lambda i: (0, 0)),
            pl.BlockSpec((R, D0), lambda i: (i, 0)),
        ],
        out_specs=pl.BlockSpec((R, D0), lambda i: (i, 0)),
        out_shape=jax.ShapeDtypeStruct((N, D0), jnp.float32),
    )(acc, den_r, S, W2, W2s, b1, W3, H4)


def kernel(S_u, S_b, edge_index_u, edge_index_b0, edge_index_b1, edge_index_b2,
           user_idx, biz_idx, W1_u, W1_b, a_u, a_b, omega, W2_u, W2_us, b1_u,
           W2_b, W2_bs, b1_b, W3_u, W3_b, H4_u, H4_b, bias_u_w, bias_b_w,
           bias_global):
    H1_u, pdps_u = _t1(S_u, W1_u, a_u)
    H1_b, pdps_b = _t1(S_b, W1_b, a_b)

    acc_u, den_u = _su(edge_index_u, H1_u, pdps_u)
    U_all = _t2(acc_u, den_u, S_u, W2_u, W2_us, b1_u, W3_u, H4_u)

    omega_s = jax.nn.softmax(omega)
    om16 = jnp.zeros((16,), jnp.float32).at[:3].set(omega_s)
    src_flat = jnp.concatenate([edge_index_b0[0], edge_index_b1[0],
                                edge_index_b2[0]])
    dst_flat = jnp.concatenate([edge_index_b0[1], edge_index_b1[1],
                                edge_index_b2[1]])
    pdp_b = jnp.pad(pdps_b, ((0, 0), (0, NUP - NU)))
    qd, qs = pdp_b[0], pdp_b[1]
    cnt = _sb1(src_flat, dst_flat)
    acc2, den2, susp, scnt = _sb2(src_flat, dst_flat, cnt, H1_b, qd, qs, om16)
    acc3, den3 = _sb3(susp, scnt, H1_b, qd, qs, om16)
    accb = jnp.concatenate([acc2, acc3])[:, :NU, :]
    denb = jnp.concatenate([den2, den3]).reshape(4, NUP)[:, :NU]
    B_all = _t2(accb, denb, S_b, W2_b, W2_bs, b1_b, W3_b, H4_b)

    U_q = U_all[user_idx]
    B_q = B_all[biz_idx]
    logit = ((U_q * B_q).sum(axis=1) + bias_u_w[user_idx, 0]
             + bias_b_w[biz_idx, 0] + bias_global[0])
    pred = (R_MAX - R_MIN) * jax.nn.sigmoid(logit) + R_MIN
    return (pred, U_all, B_all)


# speedup vs baseline: 1.8247x; 1.0858x over previous
import functools

import jax
import jax.numpy as jnp
import numpy as np
from jax import lax
from jax.experimental import pallas as pl
from jax.experimental.pallas import tpu as pltpu
from jax.experimental.pallas import tpu_sc as plsc

NU = 10000
NB = 10000
D0 = 64
EU = 320000
EB = 160000
R_MIN = 1.0
R_MAX = 5.0

NC = 2
NS = 16
NW = NC * NS


def _leaky(x):
    return jnp.where(x > 0, x, 0.2 * x)


def _t1_body(s_ref, w1_ref, h1_ref):
    h1_ref[...] = lax.dot_general(s_ref[...], w1_ref[...],
                                  (((1,), (1,)), ((), ())),
                                  preferred_element_type=jnp.float32)


def _pdps_body(a2_ref, h1_ref, pdps_ref):
    pdps_ref[...] = lax.dot_general(a2_ref[...], h1_ref[...],
                                    (((1,), (1,)), ((), ())),
                                    preferred_element_type=jnp.float32)


def _t1(S, W1, a):
    N, SD = S.shape
    a2 = jnp.stack([a[:D0], a[D0:]])
    R = 2000
    h1 = pl.pallas_call(
        _t1_body,
        grid=(N // R,),
        in_specs=[
            pl.BlockSpec((R, SD), lambda i: (i, 0)),
            pl.BlockSpec((D0, SD), lambda i: (0, 0)),
        ],
        out_specs=pl.BlockSpec((R, D0), lambda i: (i, 0)),
        out_shape=jax.ShapeDtypeStruct((N, D0), jnp.float32),
    )(S, W1)
    pdps = pl.pallas_call(
        _pdps_body,
        out_shape=jax.ShapeDtypeStruct((2, N), jnp.float32),
    )(a2, h1)
    return h1, pdps


_B = 128
NUP = 10240


def _emit_rows(h1_ref, acc_s, den_s, src_i, dst_i, rows_v, w_v):
    pltpu.sync_copy(h1_ref.at[src_i], rows_v)

    def scale_row(e, _):
        idx_e = jnp.zeros((16,), jnp.int32) + e
        w16 = plsc.load_gather(w_v, [idx_e])
        for c in range(D0 // 16):
            rows_v[e, pl.ds(c * 16, 16)] = rows_v[e, pl.ds(c * 16, 16)] * w16
        return 0

    lax.fori_loop(0, _B, scale_row, 0, unroll=2)

    pltpu.sync_copy(rows_v, acc_s.at[dst_i], add=True)
    pltpu.sync_copy(w_v, den_s.at[dst_i], add=True)


def _edge_block(ei_ref, h1_ref, pd_tab, ps_tab, acc_s, den_s,
                src_i, dst_i, rows_v, w_v, blk):
    base = pl.multiple_of(blk * 2 * _B, _B)
    pltpu.sync_copy(ei_ref.at[pl.ds(base, _B)], src_i)
    pltpu.sync_copy(ei_ref.at[pl.ds(base + _B, _B)], dst_i)

    def scores(i, _):
        s16 = src_i[pl.ds(i * 16, 16)]
        d16 = dst_i[pl.ds(i * 16, 16)]
        pdv = plsc.load_gather(pd_tab, [d16])
        psv = plsc.load_gather(ps_tab, [s16])
        w_v[pl.ds(i * 16, 16)] = jnp.exp(_leaky(pdv + psv))
        return 0

    lax.fori_loop(0, _B // 16, scores, 0, unroll=4)
    _emit_rows(h1_ref, acc_s, den_s, src_i, dst_i, rows_v, w_v)


_SS = 4


def _edge_superblock(ei_ref, h1_ref, pd_tab, ps_tab, acc_s, den_s,
                     idx_v, dst2, rows4, w4, gsem, ssem, sblk0):
    base = pl.multiple_of(sblk0 * 2 * _B, _B)
    pltpu.sync_copy(ei_ref.at[pl.ds(base, 2 * _B * _SS)], idx_v)

    gathers = []
    for k in range(_SS):
        gathers.append(pltpu.async_copy(
            h1_ref.at[idx_v.at[pl.ds(k * 2 * _B, _B)]], rows4.at[k], gsem))

    waits = []
    for k in range(_SS):
        def chunkk(i, _, k=k):
            s16 = idx_v[pl.ds(k * 2 * _B + i * 16, 16)]
            d16 = idx_v[pl.ds(k * 2 * _B + _B + i * 16, 16)]
            dst2[k, pl.ds(i * 16, 16)] = d16
            pdv = plsc.load_gather(pd_tab, [d16])
            psv = plsc.load_gather(ps_tab, [s16])
            w4[k, pl.ds(i * 16, 16)] = jnp.exp(_leaky(pdv + psv))
            return 0

        lax.fori_loop(0, _B // 16, chunkk, 0, unroll=4)
        gathers[k].wait()

        def scale_row(e, _, k=k):
            idx_e = jnp.zeros((16,), jnp.int32) + e
            w16 = plsc.load_gather(w4.at[k], [idx_e])
            for c in range(D0 // 16):
                rows4[k, e, pl.ds(c * 16, 16)] = (
                    rows4[k, e, pl.ds(c * 16, 16)] * w16)
            return 0

        lax.fori_loop(0, _B, scale_row, 0, unroll=2)
        waits.append(pltpu.async_copy(rows4.at[k], acc_s.at[dst2.at[k]],
                                      ssem, add=True))
        waits.append(pltpu.async_copy(w4.at[k], den_s.at[dst2.at[k]],
                                      ssem, add=True))

    for wt in waits:
        wt.wait()


def _part_copy(src, dst, sid):
    base = pl.multiple_of(sid * 640, 128)
    pltpu.sync_copy(src.at[pl.ds(base, 640)], dst.at[pl.ds(base, 640)])


def _su_kernel(ei_ref, h1_ref, pd_ref, ps_ref, z64_ref, z1_ref,
               acc_out, den_out,
               pd_tab, ps_tab, src_i, dst_i, rows_v, w_v,
               idx_v, dst2, rows4, w4, gsem, ssem, acc_s, den_s):
    cid = lax.axis_index("c")
    sid = lax.axis_index("s")
    wid = cid * NS + sid

    pltpu.sync_copy(pd_ref, pd_tab)
    pltpu.sync_copy(ps_ref, ps_tab)
    _part_copy(z64_ref, acc_s, sid)
    pltpu.sync_copy(z1_ref.at[pl.ds(pl.multiple_of(sid * 640, 128), 640)],
                    den_s.at[pl.ds(pl.multiple_of(sid * 640, 128), 640)])
    plsc.subcore_barrier()

    total_blk = EU // _B
    nbase = total_blk // NW
    extra = total_blk - nbase * NW
    nblk = nbase + jnp.where(wid < extra, 1, 0)
    sblk = nbase * wid + jnp.minimum(wid, extra)
    nsup = nblk // _SS

    def sup(j, _):
        _edge_superblock(ei_ref, h1_ref, pd_tab, ps_tab, acc_s, den_s,
                         idx_v, dst2, rows4, w4, gsem, ssem, sblk + j * _SS)
        return 0

    lax.fori_loop(0, nsup, sup, 0)

    def blk(b, _):
        _edge_block(ei_ref, h1_ref, pd_tab, ps_tab, acc_s, den_s,
                    src_i, dst_i, rows_v, w_v, sblk + b)
        return 0

    lax.fori_loop(nsup * _SS, nblk, blk, 0)

    plsc.subcore_barrier()
    _part_copy(acc_s, acc_out.at[cid], sid)
    base = pl.multiple_of(sid * 640, 128)
    obase = pl.multiple_of(cid * NUP + sid * 640, 128)
    pltpu.sync_copy(den_s.at[pl.ds(base, 640)],
                    den_out.at[pl.ds(obase, 640)])


def _su(edge_index, h1, pdps):
    mesh = plsc.VectorSubcoreMesh(core_axis_name="c", subcore_axis_name="s")
    z64 = jnp.zeros((NUP, D0), jnp.float32)
    z1 = jnp.zeros((NUP,), jnp.float32)
    pdp = jnp.pad(pdps, ((0, 0), (0, NUP - NU)))
    f = functools.partial(
        pl.kernel,
        mesh=mesh,
        compiler_params=pltpu.CompilerParams(needs_layout_passes=False,
                                             use_tc_tiling_on_sc=False),
        out_type=[
            jax.ShapeDtypeStruct((NC, NUP, D0), jnp.float32),
            jax.ShapeDtypeStruct((NC * NUP,), jnp.float32),
        ],
        scratch_types=[
            pltpu.VMEM((NUP,), jnp.float32),
            pltpu.VMEM((NUP,), jnp.float32),
            pltpu.VMEM((_B,), jnp.int32),
            pltpu.VMEM((_B,), jnp.int32),
            pltpu.VMEM((_B, D0), jnp.float32),
            pltpu.VMEM((_B,), jnp.float32),
            pltpu.VMEM((2 * _B * _SS,), jnp.int32),
            pltpu.VMEM((_SS, _B), jnp.int32),
            pltpu.VMEM((_SS, _B, D0), jnp.float32),
            pltpu.VMEM((_SS, _B), jnp.float32),
            pltpu.SemaphoreType.DMA,
            pltpu.SemaphoreType.DMA,
            pltpu.VMEM_SHARED((NUP, D0), jnp.float32),
            pltpu.VMEM_SHARED((NUP,), jnp.float32),
        ],
    )(_su_kernel)
    ei = edge_index.reshape(2, EU // _B, _B).swapaxes(0, 1).reshape(-1)
    acc, den = f(ei, h1, pdp[0], pdp[1], z64, z1)
    return acc[:, :NU, :], den.reshape(NC, NUP)[:, :NU]


EB3 = 3 * EB
_MHALF = 2_000_000
_MTOT = 2 * _MHALF
_DUMP = 2048
_K1 = np.int32(-1640531527)
_K2 = np.int32(-2048144789)
_K3 = np.int32(-1028477371)
_MASK31 = np.int32(0x7FFFFFFF)
_TS = 16384


def _slot_of(key):
    return ((key * _K1) & _MASK31) % _MTOT


def _cls_of(key):
    return ((key * _K2) & _MASK31) >> 26


def _probe_of(key):
    return ((key * _K3) & _MASK31) % _TS


def _sb1_kernel(src_ref, dst_ref, zb_ref, cnt_out,
                src_i, dst_i, h_i, ones_v, cnt_s):
    cid = lax.axis_index("c")
    sid = lax.axis_index("s")
    iota = lax.iota(jnp.int32, 16)

    csz = (_MHALF + _DUMP) // NS
    cbase = pl.multiple_of(sid * csz, 8)
    for j in range(7):
        pltpu.sync_copy(zb_ref, cnt_s.at[pl.ds(cbase + j * 16384, 16384)])
    pltpu.sync_copy(zb_ref.at[pl.ds(0, csz - 7 * 16384)],
                    cnt_s.at[pl.ds(cbase + 7 * 16384, csz - 7 * 16384)])

    def ones(i, _):
        ones_v[pl.ds(i * 16, 16)] = jnp.zeros((16,), jnp.int32) + 1
        return 0

    lax.fori_loop(0, _B // 16, ones, 0)
    plsc.subcore_barrier()

    total_blk = EB3 // _B
    nbase = total_blk // NS
    extra = total_blk - nbase * NS
    nblk = nbase + jnp.where(sid < extra, 1, 0)
    sblk = nbase * sid + jnp.minimum(sid, extra)
    half_lo = cid * _MHALF

    def blk(b, _):
        gb = sblk + b
        base = pl.multiple_of(gb * _B, _B)
        pltpu.sync_copy(src_ref.at[pl.ds(base, _B)], src_i)
        pltpu.sync_copy(dst_ref.at[pl.ds(base, _B)], dst_i)

        def chunk(i, _):
            s16 = src_i[pl.ds(i * 16, 16)]
            d16 = dst_i[pl.ds(i * 16, 16)]
            h = _slot_of(s16 * NB + d16) - half_lo
            own = (h >= 0) & (h < _MHALF)
            dump = _MHALF + ((gb + i) % (_DUMP // 16)) * 16 + iota
            h_i[pl.ds(i * 16, 16)] = jnp.where(own, h, dump)
            return 0

        lax.fori_loop(0, _B // 16, chunk, 0, unroll=4)
        pltpu.sync_copy(ones_v, cnt_s.at[h_i], add=True)
        return 0

    lax.fori_loop(0, nblk, blk, 0)
    plsc.subcore_barrier()

    wsz = _MHALF // NS
    rbase = pl.multiple_of(sid * wsz, 8)
    pltpu.sync_copy(cnt_s.at[pl.ds(rbase, wsz)],
                    cnt_out.at[pl.ds(pl.multiple_of(cid * _MHALF + sid * wsz, 8),
                                     wsz)])


def _sb1(src_flat, dst_flat):
    mesh = plsc.VectorSubcoreMesh(core_axis_name="c", subcore_axis_name="s")
    zb = jnp.zeros((16384,), jnp.int32)
    f = functools.partial(
        pl.kernel,
        mesh=mesh,
        compiler_params=pltpu.CompilerParams(needs_layout_passes=False,
                                             use_tc_tiling_on_sc=False),
        out_type=[jax.ShapeDtypeStruct((_MTOT,), jnp.int32)],
        scratch_types=[
            pltpu.VMEM((_B,), jnp.int32),
            pltpu.VMEM((_B,), jnp.int32),
            pltpu.VMEM((_B,), jnp.int32),
            pltpu.VMEM((_B,), jnp.int32),
            pltpu.VMEM_SHARED((_MHALF + _DUMP,), jnp.int32),
        ],
    )(_sb1_kernel)
    [cnt] = f(src_flat, dst_flat, zb)
    return cnt


_CAP = 15104


def _sb2_kernel(src_ref, dst_ref, cnt_ref, h1_ref, qd_ref, qs_ref, om_ref,
                z64_ref, z1_ref, acc_out, den_out, susp_out, scnt_out,
                qd_tab, qs_tab, om_tab, src_i, dst_i, h_i, cnt_i, stage,
                cbuf, rows_v, w_v, acc_s, den_s):
    cid = lax.axis_index("c")
    sid = lax.axis_index("s")
    wid = cid * NS + sid
    iota = lax.iota(jnp.int32, 16)

    pltpu.sync_copy(qd_ref, qd_tab)
    pltpu.sync_copy(qs_ref, qs_tab)
    pltpu.sync_copy(om_ref, om_tab)
    _part_copy(z64_ref, acc_s, sid)
    pltpu.sync_copy(z1_ref.at[pl.ds(pl.multiple_of(sid * 640, 128), 640)],
                    den_s.at[pl.ds(pl.multiple_of(sid * 640, 128), 640)])
    plsc.subcore_barrier()

    total_blk = EB3 // _B
    nbase = total_blk // NW
    extra = total_blk - nbase * NW
    nblk = nbase + jnp.where(wid < extra, 1, 0)
    sblk = nbase * wid + jnp.minimum(wid, extra)

    def blk(b, carry):
        cur, nout = carry
        gb = sblk + b
        base = pl.multiple_of(gb * _B, _B)
        g = gb // (EB // _B)
        omv = plsc.load_gather(om_tab, [jnp.zeros((16,), jnp.int32) + g])
        pltpu.sync_copy(src_ref.at[pl.ds(base, _B)], src_i)
        pltpu.sync_copy(dst_ref.at[pl.ds(base, _B)], dst_i)

        def hchunk(i, _):
            s16 = src_i[pl.ds(i * 16, 16)]
            d16 = dst_i[pl.ds(i * 16, 16)]
            h_i[pl.ds(i * 16, 16)] = _slot_of(s16 * NB + d16)
            return 0

        lax.fori_loop(0, _B // 16, hchunk, 0, unroll=4)
        pltpu.sync_copy(cnt_ref.at[h_i], cnt_i)

        def chunk(i, cur):
            s16 = src_i[pl.ds(i * 16, 16)]
            d16 = dst_i[pl.ds(i * 16, 16)]
            c16 = cnt_i[pl.ds(i * 16, 16)]
            key = s16 * NB + d16
            qdv = plsc.load_gather(qd_tab, [d16])
            qsv = plsc.load_gather(qs_tab, [s16])
            expe = jnp.exp(omv * _leaky(qdv + qsv))
            fast = c16 == 1
            w_v[pl.ds(i * 16, 16)] = jnp.where(fast, expe, 0.0)
            dst_i[pl.ds(i * 16, 16)] = jnp.where(fast, d16, NU + 16 + iota)
            susp = jnp.logical_not(fast)
            plsc.store_compressed(stage.at[pl.ds(cur, 16)], key * 4 + g,
                                  mask=susp)
            return cur + plsc.all_reduce_population_count(susp)[0]

        cur = lax.fori_loop(0, _B // 16, chunk, cur)
        _emit_rows(h1_ref, acc_s, den_s, src_i, dst_i, rows_v, w_v)

        do_flush = cur >= _B

        @pl.when(do_flush)
        def _():
            obase = pl.multiple_of(wid * _CAP + nout, _B)
            pltpu.sync_copy(stage.at[pl.ds(0, _B)],
                            susp_out.at[pl.ds(obase, _B)])
            for j in range(_B // 16):
                stage[pl.ds(j * 16, 16)] = stage[pl.ds(_B + j * 16, 16)]

        cur = jnp.where(do_flush, cur - _B, cur)
        nout = jnp.where(do_flush, nout + _B, nout)
        return (cur, nout)

    cur, nout = lax.fori_loop(0, nblk, blk,
                              (jnp.int32(0), jnp.int32(0)))

    @pl.when(cur > 0)
    def _():
        obase = pl.multiple_of(wid * _CAP + nout, _B)
        pltpu.sync_copy(stage.at[pl.ds(0, _B)], susp_out.at[pl.ds(obase, _B)])

    total = cur + nout
    plsc.store_scatter(cbuf, [iota * 0], jnp.zeros((16,), jnp.int32) + total,
                       mask=iota == 0)
    pltpu.sync_copy(cbuf.at[pl.ds(0, 8)],
                    scnt_out.at[pl.ds(pl.multiple_of(wid * 8, 8), 8)])

    plsc.subcore_barrier()
    _part_copy(acc_s, acc_out.at[cid], sid)
    base = pl.multiple_of(sid * 640, 128)
    obase = pl.multiple_of(cid * NUP + sid * 640, 128)
    pltpu.sync_copy(den_s.at[pl.ds(base, 640)], den_out.at[pl.ds(obase, 640)])


def _sb2(src_flat, dst_flat, cnt, h1b, qd, qs, om16):
    mesh = plsc.VectorSubcoreMesh(core_axis_name="c", subcore_axis_name="s")
    z64 = jnp.zeros((NUP, D0), jnp.float32)
    z1 = jnp.zeros((NUP,), jnp.float32)
    f = functools.partial(
        pl.kernel,
        mesh=mesh,
        compiler_params=pltpu.CompilerParams(needs_layout_passes=False,
                                             use_tc_tiling_on_sc=False),
        out_type=[
            jax.ShapeDtypeStruct((NC, NUP, D0), jnp.float32),
            jax.ShapeDtypeStruct((NC * NUP,), jnp.float32),
            jax.ShapeDtypeStruct((NW * _CAP,), jnp.int32),
            jax.ShapeDtypeStruct((NW * 8,), jnp.int32),
        ],
        scratch_types=[
            pltpu.VMEM((NUP,), jnp.float32),
            pltpu.VMEM((NUP,), jnp.float32),
            pltpu.VMEM((16,), jnp.float32),
            pltpu.VMEM((_B,), jnp.int32),
            pltpu.VMEM((_B,), jnp.int32),
            pltpu.VMEM((_B,), jnp.int32),
            pltpu.VMEM((_B,), jnp.int32),
            pltpu.VMEM((2 * _B + 16,), jnp.int32),
            pltpu.VMEM((16,), jnp.int32),
            pltpu.VMEM((_B, D0), jnp.float32),
            pltpu.VMEM((_B,), jnp.float32),
            pltpu.VMEM_SHARED((NUP, D0), jnp.float32),
            pltpu.VMEM_SHARED((NUP,), jnp.float32),
        ],
    )(_sb2_kernel)
    return f(src_flat, dst_flat, cnt, h1b, qd, qs, om16, z64, z1)


_SCAN = 4096


def _sb3_kernel(susp_ref, cnt_ref, h1_ref, qd_ref, qs_ref, om_ref,
                z64_ref, z1_ref, acc_out, den_out,
                qd_tab, qs_tab, om_tab, scan_v, cnts_v, src_i, dst_i,
                rows_v, w_v, tk, tw, touched, scnt_ref, acc_s, den_s):
    cid = lax.axis_index("c")
    sid = lax.axis_index("s")
    wid = cid * NS + sid
    iota = lax.iota(jnp.int32, 16)

    pltpu.sync_copy(qd_ref, qd_tab)
    pltpu.sync_copy(qs_ref, qs_tab)
    pltpu.sync_copy(om_ref, om_tab)
    _part_copy(z64_ref, acc_s, sid)
    pltpu.sync_copy(z1_ref.at[pl.ds(pl.multiple_of(sid * 640, 128), 640)],
                    den_s.at[pl.ds(pl.multiple_of(sid * 640, 128), 640)])

    def init_tab(i, _):
        tk[pl.ds(i * 16, 16)] = jnp.zeros((16,), jnp.int32) - 1
        tw[pl.ds(i * 16, 16)] = jnp.zeros((16,), jnp.float32)
        return 0

    lax.fori_loop(0, (_TS + 16) // 16, init_tab, 0)
    scnt_ref[0] = jnp.int32(0)
    plsc.subcore_barrier()

    def probe_one(pk):
        key = pk >> 2
        g = pk & 3
        omv = plsc.load_gather(om_tab, [jnp.zeros((16,), jnp.int32) + g])
        h0 = _probe_of(key)

        def cond(h):
            tkh = jnp.max(plsc.load_gather(tk, [jnp.zeros((16,), jnp.int32) + h]))
            return (tkh != key) & (tkh != -1)

        def step(h):
            return (h + 1) % _TS

        h = lax.while_loop(cond, step, h0)
        hv = jnp.zeros((16,), jnp.int32) + h
        lane0 = iota == 0
        tkh = jnp.max(plsc.load_gather(tk, [hv]))
        plsc.store_scatter(tk, [hv], jnp.zeros((16,), jnp.int32) + key, mask=lane0)
        plsc.addupdate_scatter(tw, [hv], omv, mask=lane0)

        @pl.when(tkh != key)
        def _():
            cur = scnt_ref[0]
            plsc.store_scatter(touched, [jnp.zeros((16,), jnp.int32) + cur],
                               hv, mask=lane0)
            scnt_ref[0] = cur + 1

    pltpu.sync_copy(cnt_ref, cnts_v.at[pl.ds(0, NW * 8)])

    def region(r, _):
        c_r = cnts_v[pl.ds(r * 8, 16)][0]
        nb = (c_r + _SCAN - 1) // _SCAN

        def sblock(b, _):
            base = pl.multiple_of(r * _CAP + b * _SCAN, 128)
            pltpu.sync_copy(susp_ref.at[pl.ds(base, _SCAN)], scan_v)
            rem = c_r - b * _SCAN
            nch = (jnp.minimum(rem, _SCAN) + 15) // 16

            def chunk(i, _):
                pkv = scan_v[pl.ds(i * 16, 16)]
                vmask = (i * 16 + iota) < rem
                mine = vmask & (_cls_of(pkv >> 2) == wid)
                cnt = plsc.all_reduce_population_count(mine)

                @pl.when(cnt[0] > 0)
                def _():
                    mine_i = jnp.where(mine, pkv, -1)
                    for lane in range(16):
                        pk_l = mine_i[lane]

                        @pl.when(pk_l >= 0)
                        def _():
                            probe_one(pk_l)

                return 0

            lax.fori_loop(0, nch, chunk, 0)
            return 0

        lax.fori_loop(0, nb, sblock, 0)
        return 0

    lax.fori_loop(0, NW, region, 0)

    total = scnt_ref[0]
    nb_e = (total + _B - 1) // _B

    def emit_block(tb, _):
        for i in range(_B // 16):
            pos = tb * _B + i * 16
            hidx = touched[pl.ds(pos, 16)]
            hid = jnp.where(pos + iota < total, hidx, _TS)
            key = plsc.load_gather(tk, [hid])
            wv = plsc.load_gather(tw, [hid])
            valid = key >= 0
            s16 = jnp.where(valid, key // NB, i * 16 + iota)
            d16 = key % NB
            qdv = plsc.load_gather(qd_tab, [jnp.where(valid, d16, 0)])
            qsv = plsc.load_gather(qs_tab, [s16])
            expe = jnp.where(valid, jnp.exp(wv * _leaky(qdv + qsv)), 0.0)
            src_i[pl.ds(i * 16, 16)] = s16
            dst_i[pl.ds(i * 16, 16)] = jnp.where(valid, d16, NU + 16 + iota)
            w_v[pl.ds(i * 16, 16)] = expe
        _emit_rows(h1_ref, acc_s, den_s, src_i, dst_i, rows_v, w_v)
        return 0

    lax.fori_loop(0, nb_e, emit_block, 0)

    plsc.subcore_barrier()
    _part_copy(acc_s, acc_out.at[cid], sid)
    base = pl.multiple_of(sid * 640, 128)
    obase = pl.multiple_of(cid * NUP + sid * 640, 128)
    pltpu.sync_copy(den_s.at[pl.ds(base, 640)], den_out.at[pl.ds(obase, 640)])


def _sb3(susp, scnt, h1b, qd, qs, om16):
    mesh = plsc.VectorSubcoreMesh(core_axis_name="c", subcore_axis_name="s")
    z64 = jnp.zeros((NUP, D0), jnp.float32)
    z1 = jnp.zeros((NUP,), jnp.float32)
    f = functools.partial(
        pl.kernel,
        mesh=mesh,
        compiler_params=pltpu.CompilerParams(needs_layout_passes=False,
                                             use_tc_tiling_on_sc=False),
        out_type=[
            jax.ShapeDtypeStruct((NC, NUP, D0), jnp.float32),
            jax.ShapeDtypeStruct((NC * NUP,), jnp.float32),
        ],
        scratch_types=[
            pltpu.VMEM((NUP,), jnp.float32),
            pltpu.VMEM((NUP,), jnp.float32),
            pltpu.VMEM((16,), jnp.float32),
            pltpu.VMEM((_SCAN,), jnp.int32),
            pltpu.VMEM((NW * 8 + 16,), jnp.int32),
            pltpu.VMEM((_B,), jnp.int32),
            pltpu.VMEM((_B,), jnp.int32),
            pltpu.VMEM((_B, D0), jnp.float32),
            pltpu.VMEM((_B,), jnp.float32),
            pltpu.VMEM((_TS + 16,), jnp.int32),
            pltpu.VMEM((_TS + 16,), jnp.float32),
            pltpu.VMEM((_TS,), jnp.int32),
            pltpu.SMEM((8,), jnp.int32),
            pltpu.VMEM_SHARED((NUP, D0), jnp.float32),
            pltpu.VMEM_SHARED((NUP,), jnp.float32),
        ],
    )(_sb3_kernel)
    return f(susp, scnt, h1b, qd, qs, om16, z64, z1)


def _t2_body(acc_ref, den_ref, s_ref, w2_ref, w2s_ref, b1_ref, w3_ref,
             h4_ref, out_ref):
    P = acc_ref.shape[0]
    num = acc_ref[0]
    den = den_ref[0, 0] + 1e-16
    for p in range(1, P):
        num = num + acc_ref[p]
        den = den + den_ref[0, p]
    h2 = num / den[:, None]
    h3 = (lax.dot_general(h2, w2_ref[...], (((1,), (1,)), ((), ())),
                          preferred_element_type=jnp.float32)
          + lax.dot_general(s_ref[...], w2s_ref[...], (((1,), (1,)), ((), ())),
                            preferred_element_type=jnp.float32)
          + b1_ref[...])
    h3 = jnp.where(h3 > 0, h3, jnp.exp(jnp.minimum(h3, 0.0)) - 1.0)
    u = lax.dot_general(h3, w3_ref[...], (((1,), (1,)), ((), ())),
                        preferred_element_type=jnp.float32)
    out_ref[...] = jnp.maximum(u, 0.0) + h4_ref[...]


def _t2(acc, den, S, W2, W2s, b1, W3, H4):
    N, SD = S.shape
    D1 = W2.shape[0]
    P = acc.shape[0]
    R = 2000
    den_r = den.reshape(P, N // R, R).swapaxes(0, 1)
    return pl.pallas_call(
        _t2_body,
        grid=(N // R,),
        in_specs=[
            pl.BlockSpec((P, R, D0), lambda i: (0, i, 0)),
            pl.BlockSpec((1, P, R), lambda i: (i, 0, 0)),
            pl.BlockSpec((R, SD), lambda i: (i, 0)),
            pl.BlockSpec((D1, D0), lambda i: (0, 0)),
            pl.BlockSpec((D1, SD), lambda i: (0, 0)),
            pl.BlockSpec((D1,), lambda i: (0,)),
            pl.BlockSpec((D0, D1), lambda i: (0, 0)),
            pl.BlockSpec((R, D0), lambda i: (i, 0)),
        ],
        out_specs=pl.BlockSpec((R, D0), lambda i: (i, 0)),
        out_shape=jax.ShapeDtypeStruct((N, D0), jnp.float32),
    )(acc, den_r, S, W2, W2s, b1, W3, H4)


def kernel(S_u, S_b, edge_index_u, edge_index_b0, edge_index_b1, edge_index_b2,
           user_idx, biz_idx, W1_u, W1_b, a_u, a_b, omega, W2_u, W2_us, b1_u,
           W2_b, W2_bs, b1_b, W3_u, W3_b, H4_u, H4_b, bias_u_w, bias_b_w,
           bias_global):
    H1_u, pdps_u = _t1(S_u, W1_u, a_u)
    H1_b, pdps_b = _t1(S_b, W1_b, a_b)

    acc_u, den_u = _su(edge_index_u, H1_u, pdps_u)
    U_all = _t2(acc_u, den_u, S_u, W2_u, W2_us, b1_u, W3_u, H4_u)

    omega_s = jax.nn.softmax(omega)
    om16 = jnp.zeros((16,), jnp.float32).at[:3].set(omega_s)
    src_flat = jnp.concatenate([edge_index_b0[0], edge_index_b1[0],
                                edge_index_b2[0]])
    dst_flat = jnp.concatenate([edge_index_b0[1], edge_index_b1[1],
                                edge_index_b2[1]])
    pdp_b = jnp.pad(pdps_b, ((0, 0), (0, NUP - NU)))
    qd, qs = pdp_b[0], pdp_b[1]
    cnt = _sb1(src_flat, dst_flat)
    acc2, den2, susp, scnt = _sb2(src_flat, dst_flat, cnt, H1_b, qd, qs, om16)
    acc3, den3 = _sb3(susp, scnt, H1_b, qd, qs, om16)
    accb = jnp.concatenate([acc2, acc3])[:, :NU, :]
    denb = jnp.concatenate([den2, den3]).reshape(4, NUP)[:, :NU]
    B_all = _t2(accb, denb, S_b, W2_b, W2_bs, b1_b, W3_b, H4_b)

    U_q = U_all[user_idx]
    B_q = B_all[biz_idx]
    logit = ((U_q * B_q).sum(axis=1) + bias_u_w[user_idx, 0]
             + bias_b_w[biz_idx, 0] + bias_global[0])
    pred = (R_MAX - R_MIN) * jax.nn.sigmoid(logit) + R_MIN
    return (pred, U_all, B_all)


# speedup vs baseline: 2.3615x; 1.2942x over previous
import functools

import jax
import jax.numpy as jnp
import numpy as np
from jax import lax
from jax.experimental import pallas as pl
from jax.experimental.pallas import tpu as pltpu
from jax.experimental.pallas import tpu_sc as plsc

NU = 10000
NB = 10000
D0 = 64
EU = 320000
EB = 160000
R_MIN = 1.0
R_MAX = 5.0

NC = 2
NS = 16
NW = NC * NS


def _leaky(x):
    return jnp.where(x > 0, x, 0.2 * x)


def _t1_body(s_ref, w1_ref, h1_ref):
    h1_ref[...] = lax.dot_general(s_ref[...], w1_ref[...],
                                  (((1,), (1,)), ((), ())),
                                  preferred_element_type=jnp.float32)


def _pdps_body(a2_ref, h1_ref, pdps_ref):
    pdps_ref[...] = lax.dot_general(a2_ref[...], h1_ref[...],
                                    (((1,), (1,)), ((), ())),
                                    preferred_element_type=jnp.float32)


def _t1(S, W1, a):
    N, SD = S.shape
    a2 = jnp.stack([a[:D0], a[D0:]])
    R = 2000
    h1 = pl.pallas_call(
        _t1_body,
        grid=(N // R,),
        in_specs=[
            pl.BlockSpec((R, SD), lambda i: (i, 0)),
            pl.BlockSpec((D0, SD), lambda i: (0, 0)),
        ],
        out_specs=pl.BlockSpec((R, D0), lambda i: (i, 0)),
        out_shape=jax.ShapeDtypeStruct((N, D0), jnp.float32),
    )(S, W1)
    pdps = pl.pallas_call(
        _pdps_body,
        out_shape=jax.ShapeDtypeStruct((2, N), jnp.float32),
    )(a2, h1)
    return h1, pdps


_B = 128
NUP = 10240


def _emit_rows(h1_ref, acc_s, den_s, src_i, dst_i, rows_v, w_v):
    pltpu.sync_copy(h1_ref.at[src_i], rows_v)

    def scale_row(e, _):
        idx_e = jnp.zeros((16,), jnp.int32) + e
        w16 = plsc.load_gather(w_v, [idx_e])
        for c in range(D0 // 16):
            rows_v[e, pl.ds(c * 16, 16)] = rows_v[e, pl.ds(c * 16, 16)] * w16
        return 0

    lax.fori_loop(0, _B, scale_row, 0, unroll=2)

    pltpu.sync_copy(rows_v, acc_s.at[dst_i], add=True)
    pltpu.sync_copy(w_v, den_s.at[dst_i], add=True)


def _edge_block(ei_ref, h1_ref, pd_tab, ps_tab, acc_s, den_s,
                src_i, dst_i, rows_v, w_v, blk):
    base = pl.multiple_of(blk * 2 * _B, _B)
    pltpu.sync_copy(ei_ref.at[pl.ds(base, _B)], src_i)
    pltpu.sync_copy(ei_ref.at[pl.ds(base + _B, _B)], dst_i)

    def scores(i, _):
        s16 = src_i[pl.ds(i * 16, 16)]
        d16 = dst_i[pl.ds(i * 16, 16)]
        pdv = plsc.load_gather(pd_tab, [d16])
        psv = plsc.load_gather(ps_tab, [s16])
        w_v[pl.ds(i * 16, 16)] = jnp.exp(_leaky(pdv + psv))
        return 0

    lax.fori_loop(0, _B // 16, scores, 0, unroll=4)
    _emit_rows(h1_ref, acc_s, den_s, src_i, dst_i, rows_v, w_v)


_SS = 4


def _edge_superblock(ei_ref, h1_ref, pd_tab, ps_tab, acc_s, den_s,
                     idx_v, dst2, rows4, w4, gsem, ssem, sblk0):
    base = pl.multiple_of(sblk0 * 2 * _B, _B)
    pltpu.sync_copy(ei_ref.at[pl.ds(base, 2 * _B * _SS)], idx_v)

    gathers = []
    for k in range(_SS):
        gathers.append(pltpu.async_copy(
            h1_ref.at[idx_v.at[pl.ds(k * 2 * _B, _B)]], rows4.at[k], gsem))

    waits = []
    for k in range(_SS):
        def chunkk(i, _, k=k):
            s16 = idx_v[pl.ds(k * 2 * _B + i * 16, 16)]
            d16 = idx_v[pl.ds(k * 2 * _B + _B + i * 16, 16)]
            dst2[k, pl.ds(i * 16, 16)] = d16
            pdv = plsc.load_gather(pd_tab, [d16])
            psv = plsc.load_gather(ps_tab, [s16])
            w4[k, pl.ds(i * 16, 16)] = jnp.exp(_leaky(pdv + psv))
            return 0

        lax.fori_loop(0, _B // 16, chunkk, 0, unroll=4)
        gathers[k].wait()

        def scale_row(e, _, k=k):
            idx_e = jnp.zeros((16,), jnp.int32) + e
            w16 = plsc.load_gather(w4.at[k], [idx_e])
            for c in range(D0 // 16):
                rows4[k, e, pl.ds(c * 16, 16)] = (
                    rows4[k, e, pl.ds(c * 16, 16)] * w16)
            return 0

        lax.fori_loop(0, _B, scale_row, 0, unroll=2)
        waits.append(pltpu.async_copy(rows4.at[k], acc_s.at[dst2.at[k]],
                                      ssem, add=True))
        waits.append(pltpu.async_copy(w4.at[k], den_s.at[dst2.at[k]],
                                      ssem, add=True))

    for wt in waits:
        wt.wait()


def _part_copy(src, dst, sid):
    base = pl.multiple_of(sid * 640, 128)
    pltpu.sync_copy(src.at[pl.ds(base, 640)], dst.at[pl.ds(base, 640)])


def _su_kernel(ei_ref, h1_ref, pd_ref, ps_ref, z64_ref, z1_ref,
               acc_out, den_out,
               pd_tab, ps_tab, src_i, dst_i, rows_v, w_v,
               idx_v, dst2, rows4, w4, gsem, ssem, acc_s, den_s):
    cid = lax.axis_index("c")
    sid = lax.axis_index("s")
    wid = cid * NS + sid

    pltpu.sync_copy(pd_ref, pd_tab)
    pltpu.sync_copy(ps_ref, ps_tab)
    _part_copy(z64_ref, acc_s, sid)
    pltpu.sync_copy(z1_ref.at[pl.ds(pl.multiple_of(sid * 640, 128), 640)],
                    den_s.at[pl.ds(pl.multiple_of(sid * 640, 128), 640)])
    plsc.subcore_barrier()

    total_blk = EU // _B
    nbase = total_blk // NW
    extra = total_blk - nbase * NW
    nblk = nbase + jnp.where(wid < extra, 1, 0)
    sblk = nbase * wid + jnp.minimum(wid, extra)
    nsup = nblk // _SS

    def sup(j, _):
        _edge_superblock(ei_ref, h1_ref, pd_tab, ps_tab, acc_s, den_s,
                         idx_v, dst2, rows4, w4, gsem, ssem, sblk + j * _SS)
        return 0

    lax.fori_loop(0, nsup, sup, 0)

    def blk(b, _):
        _edge_block(ei_ref, h1_ref, pd_tab, ps_tab, acc_s, den_s,
                    src_i, dst_i, rows_v, w_v, sblk + b)
        return 0

    lax.fori_loop(nsup * _SS, nblk, blk, 0)

    plsc.subcore_barrier()
    _part_copy(acc_s, acc_out.at[cid], sid)
    base = pl.multiple_of(sid * 640, 128)
    obase = pl.multiple_of(cid * NUP + sid * 640, 128)
    pltpu.sync_copy(den_s.at[pl.ds(base, 640)],
                    den_out.at[pl.ds(obase, 640)])


def _su(edge_index, h1, pdps):
    mesh = plsc.VectorSubcoreMesh(core_axis_name="c", subcore_axis_name="s")
    z64 = jnp.zeros((NUP, D0), jnp.float32)
    z1 = jnp.zeros((NUP,), jnp.float32)
    pdp = jnp.pad(pdps, ((0, 0), (0, NUP - NU)))
    f = functools.partial(
        pl.kernel,
        mesh=mesh,
        compiler_params=pltpu.CompilerParams(needs_layout_passes=False,
                                             use_tc_tiling_on_sc=False),
        out_type=[
            jax.ShapeDtypeStruct((NC, NUP, D0), jnp.float32),
            jax.ShapeDtypeStruct((NC * NUP,), jnp.float32),
        ],
        scratch_types=[
            pltpu.VMEM((NUP,), jnp.float32),
            pltpu.VMEM((NUP,), jnp.float32),
            pltpu.VMEM((_B,), jnp.int32),
            pltpu.VMEM((_B,), jnp.int32),
            pltpu.VMEM((_B, D0), jnp.float32),
            pltpu.VMEM((_B,), jnp.float32),
            pltpu.VMEM((2 * _B * _SS,), jnp.int32),
            pltpu.VMEM((_SS, _B), jnp.int32),
            pltpu.VMEM((_SS, _B, D0), jnp.float32),
            pltpu.VMEM((_SS, _B), jnp.float32),
            pltpu.SemaphoreType.DMA,
            pltpu.SemaphoreType.DMA,
            pltpu.VMEM_SHARED((NUP, D0), jnp.float32),
            pltpu.VMEM_SHARED((NUP,), jnp.float32),
        ],
    )(_su_kernel)
    ei = edge_index.reshape(2, EU // _B, _B).swapaxes(0, 1).reshape(-1)
    acc, den = f(ei, h1, pdp[0], pdp[1], z64, z1)
    return acc[:, :NU, :], den.reshape(NC, NUP)[:, :NU]


EB3 = 3 * EB
_MHALF = 2_000_000
_MTOT = 2 * _MHALF
_DUMP = 2048
_K1 = np.int32(-1640531527)
_K2 = np.int32(-2048144789)
_K3 = np.int32(-1028477371)
_MASK31 = np.int32(0x7FFFFFFF)
_TS = 16384


def _slot_of(key):
    return ((key * _K1) & _MASK31) % _MTOT


def _cls_of(key):
    return ((key * _K2) & _MASK31) >> 26


def _probe_of(key):
    return ((key * _K3) & _MASK31) % _TS


def _sb1_kernel(ei_ref, zb_ref, cnt_out,
                sd_i, h_i, ones_v, cnt_s):
    cid = lax.axis_index("c")
    sid = lax.axis_index("s")
    iota = lax.iota(jnp.int32, 16)

    csz = (_MHALF + _DUMP) // NS
    cbase = pl.multiple_of(sid * csz, 8)
    for j in range(7):
        pltpu.sync_copy(zb_ref, cnt_s.at[pl.ds(cbase + j * 16384, 16384)])
    pltpu.sync_copy(zb_ref.at[pl.ds(0, csz - 7 * 16384)],
                    cnt_s.at[pl.ds(cbase + 7 * 16384, csz - 7 * 16384)])

    def ones(i, _):
        ones_v[pl.ds(i * 16, 16)] = jnp.zeros((16,), jnp.int32) + 1
        return 0

    lax.fori_loop(0, _B // 16, ones, 0)
    plsc.subcore_barrier()

    total_blk = EB3 // _B
    nbase = total_blk // NS
    extra = total_blk - nbase * NS
    nblk = nbase + jnp.where(sid < extra, 1, 0)
    sblk = nbase * sid + jnp.minimum(sid, extra)
    half_lo = cid * _MHALF

    def blk(b, _):
        gb = sblk + b
        base = pl.multiple_of(gb * 2 * _B, _B)
        pltpu.sync_copy(ei_ref.at[pl.ds(base, 2 * _B)], sd_i)

        def chunk(i, _):
            s16 = sd_i[pl.ds(i * 16, 16)]
            d16 = sd_i[pl.ds(_B + i * 16, 16)]
            h = _slot_of(s16 * NB + d16) - half_lo
            own = (h >= 0) & (h < _MHALF)
            dump = _MHALF + ((gb + i) % (_DUMP // 16)) * 16 + iota
            h_i[pl.ds(i * 16, 16)] = jnp.where(own, h, dump)
            return 0

        lax.fori_loop(0, _B // 16, chunk, 0, unroll=4)
        pltpu.sync_copy(ones_v, cnt_s.at[h_i], add=True)
        return 0

    lax.fori_loop(0, nblk, blk, 0)
    plsc.subcore_barrier()

    wsz = _MHALF // NS
    rbase = pl.multiple_of(sid * wsz, 8)
    pltpu.sync_copy(cnt_s.at[pl.ds(rbase, wsz)],
                    cnt_out.at[pl.ds(pl.multiple_of(cid * _MHALF + sid * wsz, 8),
                                     wsz)])


def _sb1(ei):
    mesh = plsc.VectorSubcoreMesh(core_axis_name="c", subcore_axis_name="s")
    zb = jnp.zeros((16384,), jnp.int32)
    f = functools.partial(
        pl.kernel,
        mesh=mesh,
        compiler_params=pltpu.CompilerParams(needs_layout_passes=False,
                                             use_tc_tiling_on_sc=False),
        out_type=[jax.ShapeDtypeStruct((_MTOT,), jnp.int32)],
        scratch_types=[
            pltpu.VMEM((2 * _B,), jnp.int32),
            pltpu.VMEM((_B,), jnp.int32),
            pltpu.VMEM((_B,), jnp.int32),
            pltpu.VMEM_SHARED((_MHALF + _DUMP,), jnp.int32),
        ],
    )(_sb1_kernel)
    [cnt] = f(ei, zb)
    return cnt


_CAP = 15104


def _sb2_kernel(ei_ref, cnt_ref, h1_ref, qd_ref, qs_ref, om_ref,
                z64_ref, z1_ref, acc_out, den_out, susp_out, scnt_out,
                qd_tab, qs_tab, om_tab, idx_v, dst2, h2, cnt2, rows4, w4,
                stage, cbuf, gsem, csem, ssem, acc_s, den_s):
    cid = lax.axis_index("c")
    sid = lax.axis_index("s")
    wid = cid * NS + sid
    iota = lax.iota(jnp.int32, 16)

    pltpu.sync_copy(qd_ref, qd_tab)
    pltpu.sync_copy(qs_ref, qs_tab)
    pltpu.sync_copy(om_ref, om_tab)
    _part_copy(z64_ref, acc_s, sid)
    pltpu.sync_copy(z1_ref.at[pl.ds(pl.multiple_of(sid * 640, 128), 640)],
                    den_s.at[pl.ds(pl.multiple_of(sid * 640, 128), 640)])
    plsc.subcore_barrier()

    total_blk = EB3 // _B
    nbase = total_blk // NW
    extra = total_blk - nbase * NW
    nblk = nbase + jnp.where(wid < extra, 1, 0)
    sblk = nbase * wid + jnp.minimum(wid, extra)
    nsup = nblk // _SS

    def subblock(k, gb, cur):
        g = gb // (EB // _B)
        omv = plsc.load_gather(om_tab, [jnp.zeros((16,), jnp.int32) + g])

        def chunk(i, cur):
            s16 = idx_v[pl.ds(k * 2 * _B + i * 16, 16)]
            d16 = idx_v[pl.ds(k * 2 * _B + _B + i * 16, 16)]
            c16 = cnt2[k, pl.ds(i * 16, 16)]
            key = s16 * NB + d16
            qdv = plsc.load_gather(qd_tab, [d16])
            qsv = plsc.load_gather(qs_tab, [s16])
            expe = jnp.exp(omv * _leaky(qdv + qsv))
            fast = c16 == 1
            w4[k, pl.ds(i * 16, 16)] = jnp.where(fast, expe, 0.0)
            dst2[k, pl.ds(i * 16, 16)] = jnp.where(fast, d16, NU + 16 + iota)
            susp = jnp.logical_not(fast)
            plsc.store_compressed(stage.at[pl.ds(cur, 16)], key * 4 + g,
                                  mask=susp)
            return cur + plsc.all_reduce_population_count(susp)[0]

        return lax.fori_loop(0, _B // 16, chunk, cur)

    def scale_k(k):
        def scale_row(e, _):
            idx_e = jnp.zeros((16,), jnp.int32) + e
            w16 = plsc.load_gather(w4.at[k], [idx_e])
            for c in range(D0 // 16):
                rows4[k, e, pl.ds(c * 16, 16)] = (
                    rows4[k, e, pl.ds(c * 16, 16)] * w16)
            return 0

        lax.fori_loop(0, _B, scale_row, 0, unroll=2)

    def flush(cur, nout):
        do_flush = cur >= _B

        @pl.when(do_flush)
        def _():
            obase = pl.multiple_of(wid * _CAP + nout, _B)
            pltpu.sync_copy(stage.at[pl.ds(0, _B)],
                            susp_out.at[pl.ds(obase, _B)])
            for j in range(_B // 16):
                stage[pl.ds(j * 16, 16)] = stage[pl.ds(_B + j * 16, 16)]

        return (jnp.where(do_flush, cur - _B, cur),
                jnp.where(do_flush, nout + _B, nout))

    def sup(j, carry):
        cur, nout = carry
        sb0 = sblk + j * _SS
        base = pl.multiple_of(sb0 * 2 * _B, _B)
        pltpu.sync_copy(ei_ref.at[pl.ds(base, 2 * _B * _SS)], idx_v)

        gath = []
        for k in range(_SS):
            gath.append(pltpu.async_copy(
                h1_ref.at[idx_v.at[pl.ds(k * 2 * _B, _B)]], rows4.at[k],
                gsem))

        for k in range(_SS):
            def hchunk(i, _, k=k):
                s16 = idx_v[pl.ds(k * 2 * _B + i * 16, 16)]
                d16 = idx_v[pl.ds(k * 2 * _B + _B + i * 16, 16)]
                h2[k, pl.ds(i * 16, 16)] = _slot_of(s16 * NB + d16)
                return 0

            lax.fori_loop(0, _B // 16, hchunk, 0, unroll=4)

        cg = [pltpu.async_copy(cnt_ref.at[h2.at[k]], cnt2.at[k], csem)
              for k in range(_SS)]

        waits = []
        for k in range(_SS):
            cg[k].wait()
            cur = subblock(k, sb0 + k, cur)
            gath[k].wait()
            scale_k(k)
            waits.append(pltpu.async_copy(rows4.at[k], acc_s.at[dst2.at[k]],
                                          ssem, add=True))
            waits.append(pltpu.async_copy(w4.at[k], den_s.at[dst2.at[k]],
                                          ssem, add=True))
            cur, nout = flush(cur, nout)

        for wt in waits:
            wt.wait()
        return (cur, nout)

    cur, nout = lax.fori_loop(0, nsup, sup, (jnp.int32(0), jnp.int32(0)))

    def blk(b, carry):
        cur, nout = carry
        gb = sblk + b
        base = pl.multiple_of(gb * 2 * _B, _B)
        pltpu.sync_copy(ei_ref.at[pl.ds(base, 2 * _B)],
                        idx_v.at[pl.ds(0, 2 * _B)])

        def hchunk(i, _):
            s16 = idx_v[pl.ds(i * 16, 16)]
            d16 = idx_v[pl.ds(_B + i * 16, 16)]
            h2[0, pl.ds(i * 16, 16)] = _slot_of(s16 * NB + d16)
            return 0

        lax.fori_loop(0, _B // 16, hchunk, 0, unroll=4)
        pltpu.sync_copy(cnt_ref.at[h2.at[0]], cnt2.at[0])
        pltpu.async_copy(h1_ref.at[idx_v.at[pl.ds(0, _B)]], rows4.at[0],
                         gsem).wait()
        cur = subblock(0, gb, cur)
        scale_k(0)
        pltpu.sync_copy(rows4.at[0], acc_s.at[dst2.at[0]], add=True)
        pltpu.sync_copy(w4.at[0], den_s.at[dst2.at[0]], add=True)
        cur, nout = flush(cur, nout)
        return (cur, nout)

    cur, nout = lax.fori_loop(nsup * _SS, nblk, blk, (cur, nout))

    @pl.when(cur > 0)
    def _():
        obase = pl.multiple_of(wid * _CAP + nout, _B)
        pltpu.sync_copy(stage.at[pl.ds(0, _B)], susp_out.at[pl.ds(obase, _B)])

    total = cur + nout
    plsc.store_scatter(cbuf, [iota * 0], jnp.zeros((16,), jnp.int32) + total,
                       mask=iota == 0)
    pltpu.sync_copy(cbuf.at[pl.ds(0, 8)],
                    scnt_out.at[pl.ds(pl.multiple_of(wid * 8, 8), 8)])

    plsc.subcore_barrier()
    _part_copy(acc_s, acc_out.at[cid], sid)
    base = pl.multiple_of(sid * 640, 128)
    obase = pl.multiple_of(cid * NUP + sid * 640, 128)
    pltpu.sync_copy(den_s.at[pl.ds(base, 640)], den_out.at[pl.ds(obase, 640)])


def _sb2(ei, cnt, h1b, qd, qs, om16):
    mesh = plsc.VectorSubcoreMesh(core_axis_name="c", subcore_axis_name="s")
    z64 = jnp.zeros((NUP, D0), jnp.float32)
    z1 = jnp.zeros((NUP,), jnp.float32)
    f = functools.partial(
        pl.kernel,
        mesh=mesh,
        compiler_params=pltpu.CompilerParams(needs_layout_passes=False,
                                             use_tc_tiling_on_sc=False),
        out_type=[
            jax.ShapeDtypeStruct((NC, NUP, D0), jnp.float32),
            jax.ShapeDtypeStruct((NC * NUP,), jnp.float32),
            jax.ShapeDtypeStruct((NW * _CAP,), jnp.int32),
            jax.ShapeDtypeStruct((NW * 8,), jnp.int32),
        ],
        scratch_types=[
            pltpu.VMEM((NUP,), jnp.float32),
            pltpu.VMEM((NUP,), jnp.float32),
            pltpu.VMEM((16,), jnp.float32),
            pltpu.VMEM((2 * _B * _SS,), jnp.int32),
            pltpu.VMEM((_SS, _B), jnp.int32),
            pltpu.VMEM((_SS, _B), jnp.int32),
            pltpu.VMEM((_SS, _B), jnp.int32),
            pltpu.VMEM((_SS, _B, D0), jnp.float32),
            pltpu.VMEM((_SS, _B), jnp.float32),
            pltpu.VMEM((2 * _B + 16,), jnp.int32),
            pltpu.VMEM((16,), jnp.int32),
            pltpu.SemaphoreType.DMA,
            pltpu.SemaphoreType.DMA,
            pltpu.SemaphoreType.DMA,
            pltpu.VMEM_SHARED((NUP, D0), jnp.float32),
            pltpu.VMEM_SHARED((NUP,), jnp.float32),
        ],
    )(_sb2_kernel)
    return f(ei, cnt, h1b, qd, qs, om16, z64, z1)


_SCAN = 4096


def _sb3_kernel(susp_ref, cnt_ref, h1_ref, qd_ref, qs_ref, om_ref,
                z64_ref, z1_ref, acc_out, den_out,
                qd_tab, qs_tab, om_tab, scan_v, cnts_v, src_i, dst_i,
                rows_v, w_v, tk, tw, touched, scnt_ref, acc_s, den_s):
    cid = lax.axis_index("c")
    sid = lax.axis_index("s")
    wid = cid * NS + sid
    iota = lax.iota(jnp.int32, 16)

    pltpu.sync_copy(qd_ref, qd_tab)
    pltpu.sync_copy(qs_ref, qs_tab)
    pltpu.sync_copy(om_ref, om_tab)
    _part_copy(z64_ref, acc_s, sid)
    pltpu.sync_copy(z1_ref.at[pl.ds(pl.multiple_of(sid * 640, 128), 640)],
                    den_s.at[pl.ds(pl.multiple_of(sid * 640, 128), 640)])

    def init_tab(i, _):
        tk[pl.ds(i * 16, 16)] = jnp.zeros((16,), jnp.int32) - 1
        tw[pl.ds(i * 16, 16)] = jnp.zeros((16,), jnp.float32)
        return 0

    lax.fori_loop(0, (_TS + 16) // 16, init_tab, 0)
    scnt_ref[0] = jnp.int32(0)
    plsc.subcore_barrier()

    def probe_one(pk):
        key = pk >> 2
        g = pk & 3
        omv = plsc.load_gather(om_tab, [jnp.zeros((16,), jnp.int32) + g])
        h0 = _probe_of(key)

        def cond(h):
            tkh = jnp.max(plsc.load_gather(tk, [jnp.zeros((16,), jnp.int32) + h]))
            return (tkh != key) & (tkh != -1)

        def step(h):
            return (h + 1) % _TS

        h = lax.while_loop(cond, step, h0)
        hv = jnp.zeros((16,), jnp.int32) + h
        lane0 = iota == 0
        tkh = jnp.max(plsc.load_gather(tk, [hv]))
        plsc.store_scatter(tk, [hv], jnp.zeros((16,), jnp.int32) + key, mask=lane0)
        plsc.addupdate_scatter(tw, [hv], omv, mask=lane0)

        @pl.when(tkh != key)
        def _():
            cur = scnt_ref[0]
            plsc.store_scatter(touched, [jnp.zeros((16,), jnp.int32) + cur],
                               hv, mask=lane0)
            scnt_ref[0] = cur + 1

    pltpu.sync_copy(cnt_ref, cnts_v.at[pl.ds(0, NW * 8)])

    def region(r, _):
        c_r = cnts_v[pl.ds(r * 8, 16)][0]
        nb = (c_r + _SCAN - 1) // _SCAN

        def sblock(b, _):
            base = pl.multiple_of(r * _CAP + b * _SCAN, 128)
            pltpu.sync_copy(susp_ref.at[pl.ds(base, _SCAN)], scan_v)
            rem = c_r - b * _SCAN
            nch = (jnp.minimum(rem, _SCAN) + 15) // 16

            def chunk(i, _):
                pkv = scan_v[pl.ds(i * 16, 16)]
                vmask = (i * 16 + iota) < rem
                mine = vmask & (_cls_of(pkv >> 2) == wid)
                cnt = plsc.all_reduce_population_count(mine)

                @pl.when(cnt[0] > 0)
                def _():
                    mine_i = jnp.where(mine, pkv, -1)
                    for lane in range(16):
                        pk_l = mine_i[lane]

                        @pl.when(pk_l >= 0)
                        def _():
                            probe_one(pk_l)

                return 0

            lax.fori_loop(0, nch, chunk, 0)
            return 0

        lax.fori_loop(0, nb, sblock, 0)
        return 0

    lax.fori_loop(0, NW, region, 0)

    total = scnt_ref[0]
    nb_e = (total + _B - 1) // _B

    def emit_block(tb, _):
        for i in range(_B // 16):
            pos = tb * _B + i * 16
            hidx = touched[pl.ds(pos, 16)]
            hid = jnp.where(pos + iota < total, hidx, _TS)
            key = plsc.load_gather(tk, [hid])
            wv = plsc.load_gather(tw, [hid])
            valid = key >= 0
            s16 = jnp.where(valid, key // NB, i * 16 + iota)
            d16 = key % NB
            qdv = plsc.load_gather(qd_tab, [jnp.where(valid, d16, 0)])
            qsv = plsc.load_gather(qs_tab, [s16])
            expe = jnp.where(valid, jnp.exp(wv * _leaky(qdv + qsv)), 0.0)
            src_i[pl.ds(i * 16, 16)] = s16
            dst_i[pl.ds(i * 16, 16)] = jnp.where(valid, d16, NU + 16 + iota)
            w_v[pl.ds(i * 16, 16)] = expe
        _emit_rows(h1_ref, acc_s, den_s, src_i, dst_i, rows_v, w_v)
        return 0

    lax.fori_loop(0, nb_e, emit_block, 0)

    plsc.subcore_barrier()
    _part_copy(acc_s, acc_out.at[cid], sid)
    base = pl.multiple_of(sid * 640, 128)
    obase = pl.multiple_of(cid * NUP + sid * 640, 128)
    pltpu.sync_copy(den_s.at[pl.ds(base, 640)], den_out.at[pl.ds(obase, 640)])


def _sb3(susp, scnt, h1b, qd, qs, om16):
    mesh = plsc.VectorSubcoreMesh(core_axis_name="c", subcore_axis_name="s")
    z64 = jnp.zeros((NUP, D0), jnp.float32)
    z1 = jnp.zeros((NUP,), jnp.float32)
    f = functools.partial(
        pl.kernel,
        mesh=mesh,
        compiler_params=pltpu.CompilerParams(needs_layout_passes=False,
                                             use_tc_tiling_on_sc=False),
        out_type=[
            jax.ShapeDtypeStruct((NC, NUP, D0), jnp.float32),
            jax.ShapeDtypeStruct((NC * NUP,), jnp.float32),
        ],
        scratch_types=[
            pltpu.VMEM((NUP,), jnp.float32),
            pltpu.VMEM((NUP,), jnp.float32),
            pltpu.VMEM((16,), jnp.float32),
            pltpu.VMEM((_SCAN,), jnp.int32),
            pltpu.VMEM((NW * 8 + 16,), jnp.int32),
            pltpu.VMEM((_B,), jnp.int32),
            pltpu.VMEM((_B,), jnp.int32),
            pltpu.VMEM((_B, D0), jnp.float32),
            pltpu.VMEM((_B,), jnp.float32),
            pltpu.VMEM((_TS + 16,), jnp.int32),
            pltpu.VMEM((_TS + 16,), jnp.float32),
            pltpu.VMEM((_TS,), jnp.int32),
            pltpu.SMEM((8,), jnp.int32),
            pltpu.VMEM_SHARED((NUP, D0), jnp.float32),
            pltpu.VMEM_SHARED((NUP,), jnp.float32),
        ],
    )(_sb3_kernel)
    return f(susp, scnt, h1b, qd, qs, om16, z64, z1)


def _t2_body(acc_ref, den_ref, s_ref, w2_ref, w2s_ref, b1_ref, w3_ref,
             h4_ref, out_ref):
    P = acc_ref.shape[0]
    num = acc_ref[0]
    den = den_ref[0, 0] + 1e-16
    for p in range(1, P):
        num = num + acc_ref[p]
        den = den + den_ref[0, p]
    h2 = num / den[:, None]
    h3 = (lax.dot_general(h2, w2_ref[...], (((1,), (1,)), ((), ())),
                          preferred_element_type=jnp.float32)
          + lax.dot_general(s_ref[...], w2s_ref[...], (((1,), (1,)), ((), ())),
                            preferred_element_type=jnp.float32)
          + b1_ref[...])
    h3 = jnp.where(h3 > 0, h3, jnp.exp(jnp.minimum(h3, 0.0)) - 1.0)
    u = lax.dot_general(h3, w3_ref[...], (((1,), (1,)), ((), ())),
                        preferred_element_type=jnp.float32)
    out_ref[...] = jnp.maximum(u, 0.0) + h4_ref[...]


def _t2(acc, den, S, W2, W2s, b1, W3, H4):
    N, SD = S.shape
    D1 = W2.shape[0]
    P = acc.shape[0]
    R = 2000
    den_r = den.reshape(P, N // R, R).swapaxes(0, 1)
    return pl.pallas_call(
        _t2_body,
        grid=(N // R,),
        in_specs=[
            pl.BlockSpec((P, R, D0), lambda i: (0, i, 0)),
            pl.BlockSpec((1, P, R), lambda i: (i, 0, 0)),
            pl.BlockSpec((R, SD), lambda i: (i, 0)),
            pl.BlockSpec((D1, D0), lambda i: (0, 0)),
            pl.BlockSpec((D1, SD), lambda i: (0, 0)),
            pl.BlockSpec((D1,), lambda i: (0,)),
            pl.BlockSpec((D0, D1), lambda i: (0, 0)),
            pl.BlockSpec((R, D0), lambda i: (i, 0)),
        ],
        out_specs=pl.BlockSpec((R, D0), lambda i: (i, 0)),
        out_shape=jax.ShapeDtypeStruct((N, D0), jnp.float32),
    )(acc, den_r, S, W2, W2s, b1, W3, H4)


def kernel(S_u, S_b, edge_index_u, edge_index_b0, edge_index_b1, edge_index_b2,
           user_idx, biz_idx, W1_u, W1_b, a_u, a_b, omega, W2_u, W2_us, b1_u,
           W2_b, W2_bs, b1_b, W3_u, W3_b, H4_u, H4_b, bias_u_w, bias_b_w,
           bias_global):
    H1_u, pdps_u = _t1(S_u, W1_u, a_u)
    H1_b, pdps_b = _t1(S_b, W1_b, a_b)

    acc_u, den_u = _su(edge_index_u, H1_u, pdps_u)
    U_all = _t2(acc_u, den_u, S_u, W2_u, W2_us, b1_u, W3_u, H4_u)

    omega_s = jax.nn.softmax(omega)
    om16 = jnp.zeros((16,), jnp.float32).at[:3].set(omega_s)
    src_flat = jnp.concatenate([edge_index_b0[0], edge_index_b1[0],
                                edge_index_b2[0]])
    dst_flat = jnp.concatenate([edge_index_b0[1], edge_index_b1[1],
                                edge_index_b2[1]])
    pdp_b = jnp.pad(pdps_b, ((0, 0), (0, NUP - NU)))
    qd, qs = pdp_b[0], pdp_b[1]
    eib = (jnp.stack([src_flat, dst_flat])
           .reshape(2, EB3 // _B, _B).swapaxes(0, 1).reshape(-1))
    cnt = _sb1(eib)
    acc2, den2, susp, scnt = _sb2(eib, cnt, H1_b, qd, qs, om16)
    acc3, den3 = _sb3(susp, scnt, H1_b, qd, qs, om16)
    accb = jnp.concatenate([acc2, acc3])[:, :NU, :]
    denb = jnp.concatenate([den2, den3]).reshape(4, NUP)[:, :NU]
    B_all = _t2(accb, denb, S_b, W2_b, W2_bs, b1_b, W3_b, H4_b)

    U_q = U_all[user_idx]
    B_q = B_all[biz_idx]
    logit = ((U_q * B_q).sum(axis=1) + bias_u_w[user_idx, 0]
             + bias_b_w[biz_idx, 0] + bias_global[0])
    pred = (R_MAX - R_MIN) * jax.nn.sigmoid(logit) + R_MIN
    return (pred, U_all, B_all)
